# Initial kernel scaffold; baseline (speedup 1.0000x reference)
import jax
import jax.numpy as jnp
from jax import lax
from jax.experimental import pallas as pl
from jax.experimental.pallas import tpu as pltpu
from jax.experimental.pallas import tpu_sc as plsc

N = 10000
E = 160000
F_IN = 78
H1 = 10
C1 = 78
CP = 80
OUT_DIM = 128
B = 128
L = 1000
VOCAB = 27
EMB = 128

NC = 2
NS = 16
NW = NC * NS
KC = 128
NCHUNK = E // KC
ROWS_PER_TILE = N // NS

_MESH = plsc.VectorSubcoreMesh(core_axis_name="c", subcore_axis_name="s")


def _iota16():
    return lax.iota(jnp.int32, 16)


def _splat(i):
    return jnp.full((16,), i, jnp.int32)


def _lrelu_exp(v):
    return jnp.exp(jnp.maximum(v, 0.2 * v))


def _k1_body(x_ref, w_ref, as_ref, ad_ref, xp_ref, a1_ref):
    x = x_ref[...]
    w = w_ref[...]
    xp = jnp.dot(x, w, preferred_element_type=jnp.float32)
    r = x.shape[0]
    srcs = []
    dsts = []
    for h in range(H1):
        xph = xp[:, h * C1:(h + 1) * C1]
        xp_ref[h, :, 0:C1] = xph
        xp_ref[h, :, C1:CP] = jnp.zeros((r, CP - C1), jnp.float32)
        srcs.append(jnp.sum(xph * as_ref[h, :][None, :], axis=1, keepdims=True))
        dsts.append(jnp.sum(xph * ad_ref[h, :][None, :], axis=1, keepdims=True))
    z6 = jnp.zeros((r, 6), jnp.float32)
    a1_ref[...] = jnp.concatenate(srcs + [z6] + dsts + [z6], axis=1)


def _k1(x, gat1_W, gat1_as, gat1_ad):
    nblk = 10
    r = N // nblk
    return pl.pallas_call(
        _k1_body,
        grid=(nblk,),
        in_specs=[
            pl.BlockSpec((r, F_IN), lambda i: (i, 0)),
            pl.BlockSpec((F_IN, H1 * C1), lambda i: (0, 0)),
            pl.BlockSpec((H1, C1), lambda i: (0, 0)),
            pl.BlockSpec((H1, C1), lambda i: (0, 0)),
        ],
        out_specs=[
            pl.BlockSpec((H1, r, CP), lambda i: (0, i, 0)),
            pl.BlockSpec((r, 32), lambda i: (i, 0)),
        ],
        out_shape=[
            jax.ShapeDtypeStruct((H1, N, CP), jnp.float32),
            jax.ShapeDtypeStruct((N, 32), jnp.float32),
        ],
    )(x, gat1_W, gat1_as, gat1_ad)


def _s1_body(ei, a1, w1, s1p, srcv, dstv, asbuf, adbuf, wtbuf, sbuf, zbuf,
             s_acc, sem1, sem2):
    cid = lax.axis_index("c")
    sid = lax.axis_index("s")
    wid = sid * NC + cid

    def _z(i, _):
        zbuf[i, :] = jnp.zeros((16,), jnp.float32)
        return 0
    lax.fori_loop(0, ROWS_PER_TILE, _z, 0)
    pltpu.sync_copy(zbuf, s_acc.at[pl.ds(sid * ROWS_PER_TILE, ROWS_PER_TILE)])
    plsc.subcore_barrier()

    nloop = (NCHUNK + NW - 1) // NW

    def _chunk(j, _):
        cidx = wid + NW * j

        @pl.when(cidx < NCHUNK)
        def _():
            e0 = cidx * KC
            pltpu.sync_copy(ei.at[0, pl.ds(e0, KC)], srcv)
            pltpu.sync_copy(ei.at[1, pl.ds(e0, KC)], dstv)
            cp1 = pltpu.async_copy(a1.at[srcv], asbuf, sem1)
            cp2 = pltpu.async_copy(a1.at[dstv], adbuf, sem2)
            cp1.wait()
            cp2.wait()

            def _zs(i, _):
                sbuf[i, :] = jnp.zeros((16,), jnp.float32)
                return 0
            lax.fori_loop(0, KC, _zs, 0)

            it = _iota16()
            for g in range(KC // 16):
                rows = it + (16 * g)
                for h in range(H1):
                    va = plsc.load_gather(asbuf, [rows, _splat(h)])
                    vb = plsc.load_gather(adbuf, [rows, _splat(16 + h)])
                    wv = _lrelu_exp(va + vb)
                    wtbuf[h, pl.ds(16 * g, 16)] = wv
                    plsc.store_scatter(sbuf, [rows, _splat(h)], wv)
            for h in range(H1):
                pltpu.sync_copy(wtbuf.at[h, pl.ds(0, KC)],
                                w1.at[h, pl.ds(e0, KC)])
            pltpu.sync_copy(sbuf, s_acc.at[dstv], add=True)
        return 0

    lax.fori_loop(0, nloop, _chunk, 0)
    plsc.subcore_barrier()
    r0 = sid * ROWS_PER_TILE
    pltpu.sync_copy(s_acc.at[pl.ds(r0, ROWS_PER_TILE)],
                    s1p.at[cid, pl.ds(r0, ROWS_PER_TILE)])


def _s1(ei, a1):
    return pl.kernel(
        _s1_body,
        out_type=[
            jax.ShapeDtypeStruct((H1, E), jnp.float32),
            jax.ShapeDtypeStruct((NC, N, 16), jnp.float32),
        ],
        mesh=_MESH,
        scratch_types=[
            pltpu.VMEM((KC,), jnp.int32),
            pltpu.VMEM((KC,), jnp.int32),
            pltpu.VMEM((KC, 32), jnp.float32),
            pltpu.VMEM((KC, 32), jnp.float32),
            pltpu.VMEM((16, KC), jnp.float32),
            pltpu.VMEM((KC, 16), jnp.float32),
            pltpu.VMEM((ROWS_PER_TILE, 16), jnp.float32),
            pltpu.VMEM_SHARED((N, 16), jnp.float32),
            pltpu.SemaphoreType.DMA,
            pltpu.SemaphoreType.DMA,
        ],
    )(ei, a1)


def _s2_body(xp_flat, w1, ei, msg1, srcv, dstv, gidx, rowbuf, wbuf, zbuf,
             acc, sem1):
    cid = lax.axis_index("c")
    sid = lax.axis_index("s")

    def _z(i, _):
        for p in range(CP // 16):
            zbuf[i, pl.ds(16 * p, 16)] = jnp.zeros((16,), jnp.float32)
        return 0
    lax.fori_loop(0, ROWS_PER_TILE, _z, 0)

    r0 = sid * ROWS_PER_TILE
    for hl in range(H1 // NC):
        h = cid * (H1 // NC) + hl
        pltpu.sync_copy(zbuf, acc.at[pl.ds(r0, ROWS_PER_TILE)])
        plsc.subcore_barrier()

        nloop = (NCHUNK + NS - 1) // NS

        def _chunk(j, _):
            cidx = sid + NS * j

            @pl.when(cidx < NCHUNK)
            def _():
                e0 = cidx * KC
                pltpu.sync_copy(ei.at[0, pl.ds(e0, KC)], srcv)
                pltpu.sync_copy(ei.at[1, pl.ds(e0, KC)], dstv)
                base = h * N
                for g in range(KC // 16):
                    gidx[pl.ds(16 * g, 16)] = srcv[pl.ds(16 * g, 16)] + base
                pltpu.async_copy(xp_flat.at[gidx], rowbuf, sem1).wait()
                pltpu.sync_copy(w1.at[h, pl.ds(e0, KC)], wbuf)

                def _edge(i, _):
                    ws = plsc.load_gather(wbuf, [_splat(i)])
                    for p in range(CP // 16):
                        rowbuf[i, pl.ds(16 * p, 16)] = (
                            rowbuf[i, pl.ds(16 * p, 16)] * ws)
                    return 0
                lax.fori_loop(0, KC, _edge, 0)
                pltpu.sync_copy(rowbuf, acc.at[dstv], add=True)
            return 0

        lax.fori_loop(0, nloop, _chunk, 0)
        plsc.subcore_barrier()
        pltpu.sync_copy(acc.at[pl.ds(r0, ROWS_PER_TILE)],
                        msg1.at[h, pl.ds(r0, ROWS_PER_TILE)])
        plsc.subcore_barrier()


def _s2(xp_flat, w1, ei):
    return pl.kernel(
        _s2_body,
        out_type=jax.ShapeDtypeStruct((H1, N, CP), jnp.float32),
        mesh=_MESH,
        scratch_types=[
            pltpu.VMEM((KC,), jnp.int32),
            pltpu.VMEM((KC,), jnp.int32),
            pltpu.VMEM((KC,), jnp.int32),
            pltpu.VMEM((KC, CP), jnp.float32),
            pltpu.VMEM((KC,), jnp.float32),
            pltpu.VMEM((ROWS_PER_TILE, CP), jnp.float32),
            pltpu.VMEM_SHARED((N, CP), jnp.float32),
            pltpu.SemaphoreType.DMA,
        ],
    )(xp_flat, w1, ei)


def _k2_body(msg1_ref, s1p_ref, a1_ref, xp_ref, b1_ref, w2_ref, as2_ref,
             ad2_ref, hp2_ref, a2_ref):
    a1s = a1_ref[:, 0:H1]
    a1d = a1_ref[:, 16:16 + H1]
    wself = _lrelu_exp(a1s + a1d)
    s1 = s1p_ref[0, :, 0:H1] + s1p_ref[1, :, 0:H1] + wself
    parts = []
    for h in range(H1):
        msg = msg1_ref[h, :, 0:C1]
        xph = xp_ref[h, :, 0:C1]
        num = msg + wself[:, h:h + 1] * xph
        hv = num / s1[:, h:h + 1] + b1_ref[0, h * C1:(h + 1) * C1][None, :]
        parts.append(jnp.where(hv > 0, hv, jnp.expm1(hv)))
    h1 = jnp.concatenate(parts, axis=1)
    hp2 = jnp.dot(h1, w2_ref[...], preferred_element_type=jnp.float32)
    hp2_ref[...] = hp2
    asrc = jnp.sum(hp2 * as2_ref[0, :][None, :], axis=1, keepdims=True)
    adst = jnp.sum(hp2 * ad2_ref[0, :][None, :], axis=1, keepdims=True)
    wself2 = _lrelu_exp(asrc + adst)
    r = hp2.shape[0]
    a2_ref[...] = jnp.concatenate(
        [asrc, adst, wself2, jnp.zeros((r, 13), jnp.float32)], axis=1)


def _k2(msg1, s1p, a1, xp, gat1_b, gat2_W, gat2_as, gat2_ad):
    nblk = 10
    r = N // nblk
    return pl.pallas_call(
        _k2_body,
        grid=(nblk,),
        in_specs=[
            pl.BlockSpec((H1, r, CP), lambda i: (0, i, 0)),
            pl.BlockSpec((NC, r, 16), lambda i: (0, i, 0)),
            pl.BlockSpec((r, 32), lambda i: (i, 0)),
            pl.BlockSpec((H1, r, CP), lambda i: (0, i, 0)),
            pl.BlockSpec((1, H1 * C1), lambda i: (0, 0)),
            pl.BlockSpec((H1 * C1, OUT_DIM), lambda i: (0, 0)),
            pl.BlockSpec((1, OUT_DIM), lambda i: (0, 0)),
            pl.BlockSpec((1, OUT_DIM), lambda i: (0, 0)),
        ],
        out_specs=[
            pl.BlockSpec((r, OUT_DIM), lambda i: (i, 0)),
            pl.BlockSpec((r, 16), lambda i: (i, 0)),
        ],
        out_shape=[
            jax.ShapeDtypeStruct((N, OUT_DIM), jnp.float32),
            jax.ShapeDtypeStruct((N, 16), jnp.float32),
        ],
    )(msg1, s1p, a1, xp, gat1_b, gat2_W, gat2_as, gat2_ad)


def _s3_body(ei, a2, w2, s2p, srcv, dstv, g1, g2, wbuf, sbuf, zbuf, s_acc,
             sem1, sem2):
    cid = lax.axis_index("c")
    sid = lax.axis_index("s")
    wid = sid * NC + cid

    def _z(i, _):
        zbuf[i, :] = jnp.zeros((16,), jnp.float32)
        return 0
    lax.fori_loop(0, ROWS_PER_TILE, _z, 0)
    pltpu.sync_copy(zbuf, s_acc.at[pl.ds(sid * ROWS_PER_TILE, ROWS_PER_TILE)])
    plsc.subcore_barrier()

    nloop = (NCHUNK + NW - 1) // NW

    def _chunk(j, _):
        cidx = wid + NW * j

        @pl.when(cidx < NCHUNK)
        def _():
            e0 = cidx * KC
            pltpu.sync_copy(ei.at[0, pl.ds(e0, KC)], srcv)
            pltpu.sync_copy(ei.at[1, pl.ds(e0, KC)], dstv)
            cp1 = pltpu.async_copy(a2.at[srcv], g1, sem1)
            cp2 = pltpu.async_copy(a2.at[dstv], g2, sem2)
            cp1.wait()
            cp2.wait()

            def _zs(i, _):
                sbuf[i, :] = jnp.zeros((16,), jnp.float32)
                return 0
            lax.fori_loop(0, KC, _zs, 0)

            it = _iota16()
            for g in range(KC // 16):
                rows = it + (16 * g)
                va = plsc.load_gather(g1, [rows, _splat(0)])
                vb = plsc.load_gather(g2, [rows, _splat(1)])
                wv = _lrelu_exp(va + vb)
                wbuf[pl.ds(16 * g, 16)] = wv
                plsc.store_scatter(sbuf, [rows, _splat(0)], wv)
            pltpu.sync_copy(wbuf, w2.at[pl.ds(e0, KC)])
            pltpu.sync_copy(sbuf, s_acc.at[dstv], add=True)
        return 0

    lax.fori_loop(0, nloop, _chunk, 0)
    plsc.subcore_barrier()
    r0 = sid * ROWS_PER_TILE
    pltpu.sync_copy(s_acc.at[pl.ds(r0, ROWS_PER_TILE)],
                    s2p.at[cid, pl.ds(r0, ROWS_PER_TILE)])


def _s3(ei, a2):
    return pl.kernel(
        _s3_body,
        out_type=[
            jax.ShapeDtypeStruct((E,), jnp.float32),
            jax.ShapeDtypeStruct((NC, N, 16), jnp.float32),
        ],
        mesh=_MESH,
        scratch_types=[
            pltpu.VMEM((KC,), jnp.int32),
            pltpu.VMEM((KC,), jnp.int32),
            pltpu.VMEM((KC, 16), jnp.float32),
            pltpu.VMEM((KC, 16), jnp.float32),
            pltpu.VMEM((KC,), jnp.float32),
            pltpu.VMEM((KC, 16), jnp.float32),
            pltpu.VMEM((ROWS_PER_TILE, 16), jnp.float32),
            pltpu.VMEM_SHARED((N, 16), jnp.float32),
            pltpu.SemaphoreType.DMA,
            pltpu.SemaphoreType.DMA,
        ],
    )(ei, a2)


def _s4_body(hp2, w2, ei, msg2p, srcv, dstv, rowbuf, wbuf, zbuf, acc, sem1):
    cid = lax.axis_index("c")
    sid = lax.axis_index("s")

    def _z(i, _):
        for p in range(OUT_DIM // 16):
            zbuf[i, pl.ds(16 * p, 16)] = jnp.zeros((16,), jnp.float32)
        return 0
    lax.fori_loop(0, ROWS_PER_TILE, _z, 0)
    r0 = sid * ROWS_PER_TILE
    pltpu.sync_copy(zbuf, acc.at[pl.ds(r0, ROWS_PER_TILE)])
    plsc.subcore_barrier()

    half = NCHUNK // NC
    nloop = (half + NS - 1) // NS

    def _chunk(j, _):
        cl = sid + NS * j

        @pl.when(cl < half)
        def _():
            cidx = cid * half + cl
            e0 = cidx * KC
            pltpu.sync_copy(ei.at[0, pl.ds(e0, KC)], srcv)
            pltpu.sync_copy(ei.at[1, pl.ds(e0, KC)], dstv)
            pltpu.async_copy(hp2.at[srcv], rowbuf, sem1).wait()
            pltpu.sync_copy(w2.at[pl.ds(e0, KC)], wbuf)

            def _edge(i, _):
                ws = plsc.load_gather(wbuf, [_splat(i)])
                for p in range(OUT_DIM // 16):
                    rowbuf[i, pl.ds(16 * p, 16)] = (
                        rowbuf[i, pl.ds(16 * p, 16)] * ws)
                return 0
            lax.fori_loop(0, KC, _edge, 0)
            pltpu.sync_copy(rowbuf, acc.at[dstv], add=True)
        return 0

    lax.fori_loop(0, nloop, _chunk, 0)
    plsc.subcore_barrier()
    pltpu.sync_copy(acc.at[pl.ds(r0, ROWS_PER_TILE)],
                    msg2p.at[cid, pl.ds(r0, ROWS_PER_TILE)])


def _s4(hp2, w2, ei):
    return pl.kernel(
        _s4_body,
        out_type=jax.ShapeDtypeStruct((NC, N, OUT_DIM), jnp.float32),
        mesh=_MESH,
        scratch_types=[
            pltpu.VMEM((KC,), jnp.int32),
            pltpu.VMEM((KC,), jnp.int32),
            pltpu.VMEM((KC, OUT_DIM), jnp.float32),
            pltpu.VMEM((KC,), jnp.float32),
            pltpu.VMEM((ROWS_PER_TILE, OUT_DIM), jnp.float32),
            pltpu.VMEM_SHARED((N, OUT_DIM), jnp.float32),
            pltpu.SemaphoreType.DMA,
        ],
    )(hp2, w2, ei)


_RCHUNK = 80
_NRCH = N // _RCHUNK


def _s5_body(msg2p, hp2, a2, s2p, batch, b2, gpart, m0c, m1c, hpc, a2c, s0c,
             s1c, bc, bbuf, gloc):
    cid = lax.axis_index("c")
    sid = lax.axis_index("s")
    wid = sid * NC + cid

    pltpu.sync_copy(b2, bbuf)

    def _zg(i, _):
        for p in range(OUT_DIM // 16):
            gloc[i, pl.ds(16 * p, 16)] = jnp.zeros((16,), jnp.float32)
        return 0
    lax.fori_loop(0, B, _zg, 0)

    nloop = (_NRCH + NW - 1) // NW

    def _chunk(j, _):
        cidx = wid + NW * j

        @pl.when(cidx < _NRCH)
        def _():
            r0 = cidx * _RCHUNK
            sl = pl.ds(r0, _RCHUNK)
            pltpu.sync_copy(msg2p.at[0, sl], m0c)
            pltpu.sync_copy(msg2p.at[1, sl], m1c)
            pltpu.sync_copy(hp2.at[sl], hpc)
            pltpu.sync_copy(a2.at[sl], a2c)
            pltpu.sync_copy(s2p.at[0, sl], s0c)
            pltpu.sync_copy(s2p.at[1, sl], s1c)
            pltpu.sync_copy(batch.at[sl], bc)

            it = _iota16()

            def _row(i, _):
                wself = plsc.load_gather(a2c, [_splat(i), _splat(2)])
                sv = (plsc.load_gather(s0c, [_splat(i), _splat(0)])
                      + plsc.load_gather(s1c, [_splat(i), _splat(0)])
                      + wself)
                rcp = 1.0 / sv
                bid = plsc.load_gather(bc, [_splat(i)])
                for p in range(OUT_DIM // 16):
                    cs = pl.ds(16 * p, 16)
                    hv = (m0c[i, cs] + m1c[i, cs] + wself * hpc[i, cs]) * rcp
                    hv = jnp.maximum(hv + bbuf[cs], 0.0)
                    cols = it + (16 * p)
                    old = plsc.load_gather(gloc, [bid, cols])
                    plsc.store_scatter(gloc, [bid, cols],
                                       jnp.maximum(old, hv))
                return 0
            lax.fori_loop(0, _RCHUNK, _row, 0)
        return 0

    lax.fori_loop(0, nloop, _chunk, 0)
    pltpu.sync_copy(gloc, gpart.at[wid])


def _s5(msg2p, hp2, a2, s2p, batch, gat2_b):
    return pl.kernel(
        _s5_body,
        out_type=jax.ShapeDtypeStruct((NW, B, OUT_DIM), jnp.float32),
        mesh=_MESH,
        scratch_types=[
            pltpu.VMEM((_RCHUNK, OUT_DIM), jnp.float32),
            pltpu.VMEM((_RCHUNK, OUT_DIM), jnp.float32),
            pltpu.VMEM((_RCHUNK, OUT_DIM), jnp.float32),
            pltpu.VMEM((_RCHUNK, 16), jnp.float32),
            pltpu.VMEM((_RCHUNK, 16), jnp.float32),
            pltpu.VMEM((_RCHUNK, 16), jnp.float32),
            pltpu.VMEM((_RCHUNK,), jnp.int32),
            pltpu.VMEM((OUT_DIM,), jnp.float32),
            pltpu.VMEM((B, OUT_DIM), jnp.float32),
        ],
    )(msg2p, hp2, a2, s2p, batch, gat2_b)


def _kcnn_body(t_ref, emb_ref, w1_ref, b1_ref, w2_ref, b2_ref, w3_ref,
               b3_ref, out_ref):
    tids = t_ref[0]
    oh = (lax.broadcasted_iota(jnp.int32, (VOCAB, L), 0) == tids
          ).astype(jnp.float32)
    e = lax.dot_general(emb_ref[...], oh, (((0,), (0,)), ((), ())),
                        preferred_element_type=jnp.float32)

    def conv(xin, w_ref, b_ref, lout):
        acc = jnp.zeros((w_ref.shape[2], lout), jnp.float32)
        for k in range(8):
            acc = acc + lax.dot_general(
                w_ref[k], xin[:, k:k + lout], (((0,), (0,)), ((), ())),
                preferred_element_type=jnp.float32)
        return jnp.maximum(acc + b_ref[0, :][:, None], 0.0)

    y1 = conv(e, w1_ref, b1_ref, L - 7)
    y2 = conv(y1, w2_ref, b2_ref, L - 14)
    y3 = conv(y2, w3_ref, b3_ref, L - 21)
    out_ref[0, :] = jnp.max(y3, axis=1)


def _kcnn(target3, emb, c1_Wk, c1_b, c2_Wk, c2_b, c3_Wk, c3_b):
    return pl.pallas_call(
        _kcnn_body,
        grid=(B,),
        in_specs=[
            pl.BlockSpec((1, 1, L), lambda i: (i, 0, 0)),
            pl.BlockSpec((VOCAB, EMB), lambda i: (0, 0)),
            pl.BlockSpec((8, EMB, 64), lambda i: (0, 0, 0)),
            pl.BlockSpec((1, 64), lambda i: (0, 0)),
            pl.BlockSpec((8, 64, 96), lambda i: (0, 0, 0)),
            pl.BlockSpec((1, 96), lambda i: (0, 0)),
            pl.BlockSpec((8, 96, 128), lambda i: (0, 0, 0)),
            pl.BlockSpec((1, 128), lambda i: (0, 0)),
        ],
        out_specs=pl.BlockSpec((1, 128), lambda i: (i, 0)),
        out_shape=jax.ShapeDtypeStruct((B, 128), jnp.float32),
    )(target3, emb, c1_Wk, c1_b, c2_Wk, c2_b, c3_Wk, c3_b)


def _khead_body(gp_ref, t_ref, fcgw, fcgb, fcpw, fcpb, w1, b1, w2, b2, w3,
                b3, ow, ob, out_ref):
    g = jnp.max(gp_ref[...], axis=0)
    g = jnp.maximum(
        jnp.dot(g, fcgw[...], preferred_element_type=jnp.float32)
        + fcgb[0, :][None, :], 0.0)
    t = jnp.maximum(
        jnp.dot(t_ref[...], fcpw[...], preferred_element_type=jnp.float32)
        + fcpb[0, :][None, :], 0.0)
    xc = jnp.concatenate([g, t], axis=1)
    xc = jnp.maximum(
        jnp.dot(xc, w1[...], preferred_element_type=jnp.float32)
        + b1[0, :][None, :], 0.0)
    xc = jnp.maximum(
        jnp.dot(xc, w2[...], preferred_element_type=jnp.float32)
        + b2[0, :][None, :], 0.0)
    xc = jnp.maximum(
        jnp.dot(xc, w3[...], preferred_element_type=jnp.float32)
        + b3[0, :][None, :], 0.0)
    out_ref[...] = (jnp.dot(xc, ow[...], preferred_element_type=jnp.float32)
                    + ob[0, :][None, :])


def _khead(gpart, t, fcg_W, fcg_b, fcp_W, fcp_b, fc1_W, fc1_b, fc2_W, fc2_b,
           fc3_W, fc3_b, outW, outb):
    return pl.pallas_call(
        _khead_body,
        out_shape=jax.ShapeDtypeStruct((B, 1), jnp.float32),
    )(gpart, t, fcg_W, fcg_b, fcp_W, fcp_b, fc1_W, fc1_b, fc2_W, fc2_b,
      fc3_W, fc3_b, outW, outb)


def kernel(x, edge_index, batch, target, gat1_W, gat1_as, gat1_ad, gat1_b,
           gat2_W, gat2_as, gat2_ad, gat2_b, fcg_W, fcg_b, emb, c1_W, c1_b,
           c2_W, c2_b, c3_W, c3_b, fcp_W, fcp_b, fc1_W, fc1_b, fc2_W, fc2_b,
           fc3_W, fc3_b, outW, outb):
    ei = edge_index

    xp, a1 = _k1(x, gat1_W, gat1_as, gat1_ad)
    w1, s1p = _s1(ei, a1)
    msg1 = _s2(xp.reshape(H1 * N, CP), w1, ei)
    hp2, a2 = _k2(msg1, s1p, a1, xp, gat1_b.reshape(1, -1), gat2_W,
                  gat2_as, gat2_ad)
    w2, s2p = _s3(ei, a2)
    msg2p = _s4(hp2, w2, ei)
    gpart = _s5(msg2p, hp2, a2, s2p, batch, gat2_b)

    t = _kcnn(target.reshape(B, 1, L), emb,
              jnp.transpose(c1_W, (2, 1, 0)), c1_b.reshape(1, -1),
              jnp.transpose(c2_W, (2, 1, 0)), c2_b.reshape(1, -1),
              jnp.transpose(c3_W, (2, 1, 0)), c3_b.reshape(1, -1))

    return _khead(gpart, t, fcg_W, fcg_b.reshape(1, -1), fcp_W,
                  fcp_b.reshape(1, -1), fc1_W, fc1_b.reshape(1, -1),
                  fc2_W, fc2_b.reshape(1, -1), fc3_W, fc3_b.reshape(1, -1),
                  outW, outb.reshape(1, -1))


# speedup vs baseline: 11.0798x; 11.0798x over previous
import jax
import jax.numpy as jnp
from jax import lax
from jax.experimental import pallas as pl
from jax.experimental.pallas import tpu as pltpu
from jax.experimental.pallas import tpu_sc as plsc

N = 10000
E = 160000
F_IN = 78
H1 = 10
C1 = 78
CP = 80
OUT_DIM = 128
B = 128
L = 1000
VOCAB = 27
EMB = 128

NC = 2
NS = 16
NW = NC * NS
KC = 128
NCHUNK = E // KC
NP = 10240
ROWS_PER_TILE = NP // NS

def _mesh():
    return plsc.VectorSubcoreMesh(core_axis_name="c", subcore_axis_name="s",
                                  num_cores=NC, num_subcores=NS)


def _iota16():
    return lax.iota(jnp.int32, 16)


def _splat(i):
    return jnp.full((16,), i, jnp.int32)


def _lrelu_exp(v):
    return jnp.exp(jnp.maximum(v, 0.2 * v))


def _k1_body(x_ref, w_ref, as_ref, ad_ref, xp_ref, a1_ref):
    x = x_ref[...]
    w = w_ref[...]
    xp = jnp.dot(x, w, preferred_element_type=jnp.float32)
    r = x.shape[0]
    srcs = []
    dsts = []
    for h in range(H1):
        xph = xp[:, h * C1:(h + 1) * C1]
        xp_ref[h, :, 0:C1] = xph
        xp_ref[h, :, C1:CP] = jnp.zeros((r, CP - C1), jnp.float32)
        srcs.append(jnp.sum(xph * as_ref[h, :][None, :], axis=1, keepdims=True))
        dsts.append(jnp.sum(xph * ad_ref[h, :][None, :], axis=1, keepdims=True))
    z6 = jnp.zeros((r, 6), jnp.float32)
    a1_ref[...] = jnp.concatenate(srcs + [z6] + dsts + [z6], axis=1)


def _k1(x, gat1_W, gat1_as, gat1_ad):
    nblk = 10
    r = N // nblk
    return pl.pallas_call(
        _k1_body,
        grid=(nblk,),
        in_specs=[
            pl.BlockSpec((r, F_IN), lambda i: (i, 0)),
            pl.BlockSpec((F_IN, H1 * C1), lambda i: (0, 0)),
            pl.BlockSpec((H1, C1), lambda i: (0, 0)),
            pl.BlockSpec((H1, C1), lambda i: (0, 0)),
        ],
        out_specs=[
            pl.BlockSpec((H1, r, CP), lambda i: (0, i, 0)),
            pl.BlockSpec((r, 32), lambda i: (i, 0)),
        ],
        out_shape=[
            jax.ShapeDtypeStruct((H1, N, CP), jnp.float32),
            jax.ShapeDtypeStruct((N, 32), jnp.float32),
        ],
    )(x, gat1_W, gat1_as, gat1_ad)


def _s1_body(ei, a1, w1, srcv, dstv, asbuf, adbuf, wtbuf, sem1, sem2):
    cid = lax.axis_index("c")
    sid = lax.axis_index("s")
    wid = sid * NC + cid

    nloop = (NCHUNK + NW - 1) // NW

    def _chunk(j, _):
        cidx = wid + NW * j

        @pl.when(cidx < NCHUNK)
        def _():
            e0 = cidx * KC
            pltpu.sync_copy(ei.at[0, pl.ds(e0, KC)], srcv)
            pltpu.sync_copy(ei.at[1, pl.ds(e0, KC)], dstv)
            cp1 = pltpu.async_copy(a1.at[srcv], asbuf, sem1)
            cp2 = pltpu.async_copy(a1.at[dstv], adbuf, sem2)
            cp1.wait()
            cp2.wait()

            it = _iota16()
            for g in range(KC // 16):
                rows = it + (16 * g)
                for h in range(H1):
                    va = plsc.load_gather(asbuf, [rows, _splat(h)])
                    vb = plsc.load_gather(adbuf, [rows, _splat(16 + h)])
                    wv = _lrelu_exp(va + vb)
                    wtbuf[h, pl.ds(16 * g, 16)] = wv
            for h in range(H1):
                pltpu.sync_copy(wtbuf.at[h, pl.ds(0, KC)],
                                w1.at[h, pl.ds(e0, KC)])
        return 0

    lax.fori_loop(0, nloop, _chunk, 0)


def _s1(ei, a1):
    return pl.kernel(
        _s1_body,
        out_type=jax.ShapeDtypeStruct((H1, E), jnp.float32),
        mesh=_mesh(),
        compiler_params=pltpu.CompilerParams(needs_layout_passes=False, use_tc_tiling_on_sc=False),
        scratch_types=[
            pltpu.VMEM((KC,), jnp.int32),
            pltpu.VMEM((KC,), jnp.int32),
            pltpu.VMEM((KC, 32), jnp.float32),
            pltpu.VMEM((KC, 32), jnp.float32),
            pltpu.VMEM((16, KC), jnp.float32),
            pltpu.SemaphoreType.DMA,
            pltpu.SemaphoreType.DMA,
        ],
    )(ei, a1)


def _s2_body(xp_flat, w1, ei, msg1, srcv, dstv, gidx, rowbuf, wbuf, zbuf,
             acc, sem1):
    cid = lax.axis_index("c")
    sid = lax.axis_index("s")

    def _z(i, _):
        for p in range(CP // 16):
            zbuf[i, pl.ds(16 * p, 16)] = jnp.zeros((16,), jnp.float32)
        return 0
    lax.fori_loop(0, ROWS_PER_TILE, _z, 0)

    r0 = sid * ROWS_PER_TILE
    for hl in range(H1 // NC):
        h = cid * (H1 // NC) + hl
        pltpu.sync_copy(zbuf, acc.at[pl.ds(r0, ROWS_PER_TILE)])
        plsc.subcore_barrier()

        nloop = (NCHUNK + NS - 1) // NS

        def _chunk(j, _):
            cidx = sid + NS * j

            @pl.when(cidx < NCHUNK)
            def _():
                e0 = cidx * KC
                pltpu.sync_copy(ei.at[0, pl.ds(e0, KC)], srcv)
                pltpu.sync_copy(ei.at[1, pl.ds(e0, KC)], dstv)
                base = h * N
                for g in range(KC // 16):
                    gidx[pl.ds(16 * g, 16)] = srcv[pl.ds(16 * g, 16)] + base
                pltpu.async_copy(xp_flat.at[gidx], rowbuf, sem1).wait()
                pltpu.sync_copy(w1.at[h, pl.ds(e0, KC)], wbuf)

                it16 = _iota16()

                def _edge(i, _):
                    ws = plsc.load_gather(wbuf, [_splat(i)])
                    for p in range(CP // 16 - 1):
                        rowbuf[i, pl.ds(16 * p, 16)] = (
                            rowbuf[i, pl.ds(16 * p, 16)] * ws)
                    pv = rowbuf[i, pl.ds(CP - 16, 16)] * ws
                    rowbuf[i, pl.ds(CP - 16, 16)] = jnp.where(
                        it16 == (C1 - (CP - 16)), ws, pv)
                    return 0
                lax.fori_loop(0, KC, _edge, 0)
                pltpu.sync_copy(rowbuf, acc.at[dstv], add=True)
            return 0

        lax.fori_loop(0, nloop, _chunk, 0)
        plsc.subcore_barrier()
        pltpu.sync_copy(acc.at[pl.ds(r0, ROWS_PER_TILE)],
                        msg1.at[h, pl.ds(r0, ROWS_PER_TILE)])
        plsc.subcore_barrier()


def _s2(xp_flat, w1, ei):
    return pl.kernel(
        _s2_body,
        out_type=jax.ShapeDtypeStruct((H1, NP, CP), jnp.float32),
        mesh=_mesh(),
        compiler_params=pltpu.CompilerParams(needs_layout_passes=False, use_tc_tiling_on_sc=False),
        scratch_types=[
            pltpu.VMEM((KC,), jnp.int32),
            pltpu.VMEM((KC,), jnp.int32),
            pltpu.VMEM((KC,), jnp.int32),
            pltpu.VMEM((KC, CP), jnp.float32),
            pltpu.VMEM((KC,), jnp.float32),
            pltpu.VMEM((ROWS_PER_TILE, CP), jnp.float32),
            pltpu.VMEM_SHARED((NP, CP), jnp.float32),
            pltpu.SemaphoreType.DMA,
        ],
    )(xp_flat, w1, ei)


def _k2_body(msg1_ref, a1_ref, xp_ref, b1_ref, w2_ref, as2_ref,
             ad2_ref, hp2_ref, a2_ref):
    a1s = a1_ref[:, 0:H1]
    a1d = a1_ref[:, 16:16 + H1]
    wself = _lrelu_exp(a1s + a1d)
    parts = []
    for h in range(H1):
        msg = msg1_ref[h, :, 0:C1]
        xph = xp_ref[h, :, 0:C1]
        sh = msg1_ref[h, :, C1:C1 + 1] + wself[:, h:h + 1]
        num = msg + wself[:, h:h + 1] * xph
        hv = num / sh + b1_ref[0, h * C1:(h + 1) * C1][None, :]
        parts.append(jnp.where(hv > 0, hv, jnp.exp(jnp.minimum(hv, 0.0)) - 1.0))
    h1 = jnp.concatenate(parts, axis=1)
    hp2 = jnp.dot(h1, w2_ref[...], preferred_element_type=jnp.float32)
    hp2_ref[...] = hp2
    asrc = jnp.sum(hp2 * as2_ref[0, :][None, :], axis=1, keepdims=True)
    adst = jnp.sum(hp2 * ad2_ref[0, :][None, :], axis=1, keepdims=True)
    wself2 = _lrelu_exp(asrc + adst)
    r = hp2.shape[0]
    a2_ref[...] = jnp.concatenate(
        [asrc, adst, wself2, jnp.zeros((r, 13), jnp.float32)], axis=1)


def _k2(msg1, a1, xp, gat1_b, gat2_W, gat2_as, gat2_ad):
    nblk = 10
    r = N // nblk
    return pl.pallas_call(
        _k2_body,
        grid=(nblk,),
        in_specs=[
            pl.BlockSpec((H1, r, CP), lambda i: (0, i, 0)),
            pl.BlockSpec((r, 32), lambda i: (i, 0)),
            pl.BlockSpec((H1, r, CP), lambda i: (0, i, 0)),
            pl.BlockSpec((1, H1 * C1), lambda i: (0, 0)),
            pl.BlockSpec((H1 * C1, OUT_DIM), lambda i: (0, 0)),
            pl.BlockSpec((1, OUT_DIM), lambda i: (0, 0)),
            pl.BlockSpec((1, OUT_DIM), lambda i: (0, 0)),
        ],
        out_specs=[
            pl.BlockSpec((r, OUT_DIM), lambda i: (i, 0)),
            pl.BlockSpec((r, 16), lambda i: (i, 0)),
        ],
        out_shape=[
            jax.ShapeDtypeStruct((N, OUT_DIM), jnp.float32),
            jax.ShapeDtypeStruct((N, 16), jnp.float32),
        ],
    )(msg1, a1, xp, gat1_b, gat2_W, gat2_as, gat2_ad)


def _s3_body(ei, a2, w2, s2p, srcv, dstv, g1, g2, wbuf, sbuf, zbuf, s_acc,
             sem1, sem2):
    cid = lax.axis_index("c")
    sid = lax.axis_index("s")
    wid = sid * NC + cid

    def _z(i, _):
        zbuf[i, :] = jnp.zeros((16,), jnp.float32)
        return 0
    lax.fori_loop(0, ROWS_PER_TILE, _z, 0)
    pltpu.sync_copy(zbuf, s_acc.at[pl.ds(sid * ROWS_PER_TILE, ROWS_PER_TILE)])
    plsc.subcore_barrier()

    nloop = (NCHUNK + NW - 1) // NW

    def _chunk(j, _):
        cidx = wid + NW * j

        @pl.when(cidx < NCHUNK)
        def _():
            e0 = cidx * KC
            pltpu.sync_copy(ei.at[0, pl.ds(e0, KC)], srcv)
            pltpu.sync_copy(ei.at[1, pl.ds(e0, KC)], dstv)
            cp1 = pltpu.async_copy(a2.at[srcv], g1, sem1)
            cp2 = pltpu.async_copy(a2.at[dstv], g2, sem2)
            cp1.wait()
            cp2.wait()

            def _zs(i, _):
                sbuf[i, :] = jnp.zeros((16,), jnp.float32)
                return 0
            lax.fori_loop(0, KC, _zs, 0)

            it = _iota16()
            for g in range(KC // 16):
                rows = it + (16 * g)
                va = plsc.load_gather(g1, [rows, _splat(0)])
                vb = plsc.load_gather(g2, [rows, _splat(1)])
                wv = _lrelu_exp(va + vb)
                wbuf[pl.ds(16 * g, 16)] = wv
                plsc.store_scatter(sbuf, [rows, _splat(0)], wv)
            pltpu.sync_copy(wbuf, w2.at[pl.ds(e0, KC)])
            pltpu.sync_copy(sbuf, s_acc.at[dstv], add=True)
        return 0

    lax.fori_loop(0, nloop, _chunk, 0)
    plsc.subcore_barrier()
    r0 = sid * ROWS_PER_TILE
    pltpu.sync_copy(s_acc.at[pl.ds(r0, ROWS_PER_TILE)],
                    s2p.at[cid, pl.ds(r0, ROWS_PER_TILE)])


def _s3(ei, a2):
    return pl.kernel(
        _s3_body,
        out_type=[
            jax.ShapeDtypeStruct((E,), jnp.float32),
            jax.ShapeDtypeStruct((NC, NP, 16), jnp.float32),
        ],
        mesh=_mesh(),
        compiler_params=pltpu.CompilerParams(needs_layout_passes=False, use_tc_tiling_on_sc=False),
        scratch_types=[
            pltpu.VMEM((KC,), jnp.int32),
            pltpu.VMEM((KC,), jnp.int32),
            pltpu.VMEM((KC, 16), jnp.float32),
            pltpu.VMEM((KC, 16), jnp.float32),
            pltpu.VMEM((KC,), jnp.float32),
            pltpu.VMEM((KC, 16), jnp.float32),
            pltpu.VMEM((ROWS_PER_TILE, 16), jnp.float32),
            pltpu.VMEM_SHARED((NP, 16), jnp.float32),
            pltpu.SemaphoreType.DMA,
            pltpu.SemaphoreType.DMA,
        ],
    )(ei, a2)


def _s4_body(hp2_pairs, w2, ei, msg2h, srcv, dstv, gidx, rowbuf, wbuf,
             zbuf, acc, sem1):
    cid = lax.axis_index("c")
    sid = lax.axis_index("s")

    def _z(i, _):
        for p in range(4):
            zbuf[i, pl.ds(16 * p, 16)] = jnp.zeros((16,), jnp.float32)
        return 0
    lax.fori_loop(0, ROWS_PER_TILE, _z, 0)
    r0 = sid * ROWS_PER_TILE
    pltpu.sync_copy(zbuf, acc.at[pl.ds(r0, ROWS_PER_TILE)])
    plsc.subcore_barrier()

    nloop = (NCHUNK + NS - 1) // NS

    def _chunk(j, _):
        cidx = sid + NS * j

        @pl.when(cidx < NCHUNK)
        def _():
            e0 = cidx * KC
            pltpu.sync_copy(ei.at[0, pl.ds(e0, KC)], srcv)
            pltpu.sync_copy(ei.at[1, pl.ds(e0, KC)], dstv)
            for g in range(KC // 16):
                gidx[pl.ds(16 * g, 16)] = srcv[pl.ds(16 * g, 16)] * 2 + cid
            pltpu.async_copy(hp2_pairs.at[gidx], rowbuf, sem1).wait()
            pltpu.sync_copy(w2.at[pl.ds(e0, KC)], wbuf)

            def _edge(i, _):
                ws = plsc.load_gather(wbuf, [_splat(i)])
                for p in range(4):
                    rowbuf[i, pl.ds(16 * p, 16)] = (
                        rowbuf[i, pl.ds(16 * p, 16)] * ws)
                return 0
            lax.fori_loop(0, KC, _edge, 0)
            pltpu.sync_copy(rowbuf, acc.at[dstv], add=True)
        return 0

    lax.fori_loop(0, nloop, _chunk, 0)
    plsc.subcore_barrier()
    pltpu.sync_copy(acc.at[pl.ds(r0, ROWS_PER_TILE)],
                    msg2h.at[cid, pl.ds(r0, ROWS_PER_TILE)])


def _s4(hp2_pairs, w2, ei):
    return pl.kernel(
        _s4_body,
        out_type=jax.ShapeDtypeStruct((NC, NP, 64), jnp.float32),
        mesh=_mesh(),
        compiler_params=pltpu.CompilerParams(needs_layout_passes=False, use_tc_tiling_on_sc=False),
        scratch_types=[
            pltpu.VMEM((KC,), jnp.int32),
            pltpu.VMEM((KC,), jnp.int32),
            pltpu.VMEM((KC,), jnp.int32),
            pltpu.VMEM((KC, 64), jnp.float32),
            pltpu.VMEM((KC,), jnp.float32),
            pltpu.VMEM((ROWS_PER_TILE, 64), jnp.float32),
            pltpu.VMEM_SHARED((NP, 64), jnp.float32),
            pltpu.SemaphoreType.DMA,
        ],
    )(hp2_pairs, w2, ei)


_RCHUNK = 80
_NRCH = N // _RCHUNK


def _s5_body(msg2h, hp2, a2, s2p, batch, b2, gpart, m0c, m1c, hpc, a2c, s0c,
             s1c, bc, bbuf, gloc):
    cid = lax.axis_index("c")
    sid = lax.axis_index("s")
    wid = sid * NC + cid

    pltpu.sync_copy(b2, bbuf)

    def _zg(i, _):
        for p in range(OUT_DIM // 16):
            gloc[i, pl.ds(16 * p, 16)] = jnp.zeros((16,), jnp.float32)
        return 0
    lax.fori_loop(0, B, _zg, 0)

    nloop = (_NRCH + NW - 1) // NW

    def _chunk(j, _):
        cidx = wid + NW * j

        @pl.when(cidx < _NRCH)
        def _():
            r0 = cidx * _RCHUNK
            sl = pl.ds(r0, _RCHUNK)
            pltpu.sync_copy(msg2h.at[0, sl], m0c)
            pltpu.sync_copy(msg2h.at[1, sl], m1c)
            pltpu.sync_copy(hp2.at[sl], hpc)
            pltpu.sync_copy(a2.at[sl], a2c)
            pltpu.sync_copy(s2p.at[0, sl], s0c)
            pltpu.sync_copy(s2p.at[1, sl], s1c)
            pltpu.sync_copy(batch.at[sl], bc)

            it = _iota16()

            def _row(i, _):
                wself = plsc.load_gather(a2c, [_splat(i), _splat(2)])
                sv = (plsc.load_gather(s0c, [_splat(i), _splat(0)])
                      + plsc.load_gather(s1c, [_splat(i), _splat(0)])
                      + wself)
                rcp = 1.0 / sv
                bid = plsc.load_gather(bc, [_splat(i)])
                for p in range(OUT_DIM // 16):
                    mc = m0c if p < 4 else m1c
                    mslice = mc[i, pl.ds(16 * (p % 4), 16)]
                    hv = (mslice + wself * hpc[i, pl.ds(16 * p, 16)]) * rcp
                    hv = jnp.maximum(hv + bbuf[pl.ds(16 * p, 16)], 0.0)
                    cols = it + (16 * p)
                    old = plsc.load_gather(gloc, [bid, cols])
                    plsc.store_scatter(gloc, [bid, cols],
                                       jnp.maximum(old, hv))
                return 0
            lax.fori_loop(0, _RCHUNK, _row, 0)
        return 0

    lax.fori_loop(0, nloop, _chunk, 0)
    pltpu.sync_copy(gloc, gpart.at[wid])


def _s5(msg2h, hp2, a2, s2p, batch, gat2_b):
    return pl.kernel(
        _s5_body,
        out_type=jax.ShapeDtypeStruct((NW, B, OUT_DIM), jnp.float32),
        mesh=_mesh(),
        compiler_params=pltpu.CompilerParams(needs_layout_passes=False, use_tc_tiling_on_sc=False),
        scratch_types=[
            pltpu.VMEM((_RCHUNK, 64), jnp.float32),
            pltpu.VMEM((_RCHUNK, 64), jnp.float32),
            pltpu.VMEM((_RCHUNK, OUT_DIM), jnp.float32),
            pltpu.VMEM((_RCHUNK, 16), jnp.float32),
            pltpu.VMEM((_RCHUNK, 16), jnp.float32),
            pltpu.VMEM((_RCHUNK, 16), jnp.float32),
            pltpu.VMEM((_RCHUNK,), jnp.int32),
            pltpu.VMEM((OUT_DIM,), jnp.float32),
            pltpu.VMEM((B, OUT_DIM), jnp.float32),
        ],
    )(msg2h, hp2, a2, s2p, batch, gat2_b)


def _kcnn_body(t_ref, emb_ref, w1_ref, b1_ref, w2_ref, b2_ref, w3_ref,
               b3_ref, out_ref):
    tids = t_ref[0]
    oh = (lax.broadcasted_iota(jnp.int32, (VOCAB, L), 0) == tids
          ).astype(jnp.float32)
    e = lax.dot_general(emb_ref[...], oh, (((0,), (0,)), ((), ())),
                        preferred_element_type=jnp.float32)

    def conv(xin, w_ref, b_ref, lout):
        acc = jnp.zeros((w_ref.shape[2], lout), jnp.float32)
        for k in range(8):
            acc = acc + lax.dot_general(
                w_ref[k], xin[:, k:k + lout], (((0,), (0,)), ((), ())),
                preferred_element_type=jnp.float32)
        return jnp.maximum(acc + b_ref[0, :][:, None], 0.0)

    y1 = conv(e, w1_ref, b1_ref, L - 7)
    y2 = conv(y1, w2_ref, b2_ref, L - 14)
    y3 = conv(y2, w3_ref, b3_ref, L - 21)
    out_ref[0, 0, :] = jnp.max(y3, axis=1)


def _kcnn(target3, emb, c1_Wk, c1_b, c2_Wk, c2_b, c3_Wk, c3_b):
    return pl.pallas_call(
        _kcnn_body,
        grid=(B,),
        in_specs=[
            pl.BlockSpec((1, 1, L), lambda i: (i, 0, 0)),
            pl.BlockSpec((VOCAB, EMB), lambda i: (0, 0)),
            pl.BlockSpec((8, EMB, 64), lambda i: (0, 0, 0)),
            pl.BlockSpec((1, 64), lambda i: (0, 0)),
            pl.BlockSpec((8, 64, 96), lambda i: (0, 0, 0)),
            pl.BlockSpec((1, 96), lambda i: (0, 0)),
            pl.BlockSpec((8, 96, 128), lambda i: (0, 0, 0)),
            pl.BlockSpec((1, 128), lambda i: (0, 0)),
        ],
        out_specs=pl.BlockSpec((1, 1, 128), lambda i: (i, 0, 0)),
        out_shape=jax.ShapeDtypeStruct((B, 1, 128), jnp.float32),
    )(target3, emb, c1_Wk, c1_b, c2_Wk, c2_b, c3_Wk, c3_b)


def _khead_body(gp_ref, t_ref, fcgw, fcgb, fcpw, fcpb, w1, b1, w2, b2, w3,
                b3, ow, ob, out_ref):
    g = jnp.max(gp_ref[...], axis=0)
    g = jnp.maximum(
        jnp.dot(g, fcgw[...], preferred_element_type=jnp.float32)
        + fcgb[0, :][None, :], 0.0)
    t = jnp.maximum(
        jnp.dot(t_ref[...], fcpw[...], preferred_element_type=jnp.float32)
        + fcpb[0, :][None, :], 0.0)
    xc = jnp.concatenate([g, t], axis=1)
    xc = jnp.maximum(
        jnp.dot(xc, w1[...], preferred_element_type=jnp.float32)
        + b1[0, :][None, :], 0.0)
    xc = jnp.maximum(
        jnp.dot(xc, w2[...], preferred_element_type=jnp.float32)
        + b2[0, :][None, :], 0.0)
    xc = jnp.maximum(
        jnp.dot(xc, w3[...], preferred_element_type=jnp.float32)
        + b3[0, :][None, :], 0.0)
    out_ref[...] = (jnp.dot(xc, ow[...], preferred_element_type=jnp.float32)
                    + ob[0, :][None, :])


def _khead(gpart, t, fcg_W, fcg_b, fcp_W, fcp_b, fc1_W, fc1_b, fc2_W, fc2_b,
           fc3_W, fc3_b, outW, outb):
    return pl.pallas_call(
        _khead_body,
        out_shape=jax.ShapeDtypeStruct((B, 1), jnp.float32),
    )(gpart, t, fcg_W, fcg_b, fcp_W, fcp_b, fc1_W, fc1_b, fc2_W, fc2_b,
      fc3_W, fc3_b, outW, outb)


def kernel(x, edge_index, batch, target, gat1_W, gat1_as, gat1_ad, gat1_b,
           gat2_W, gat2_as, gat2_ad, gat2_b, fcg_W, fcg_b, emb, c1_W, c1_b,
           c2_W, c2_b, c3_W, c3_b, fcp_W, fcp_b, fc1_W, fc1_b, fc2_W, fc2_b,
           fc3_W, fc3_b, outW, outb):
    ei = edge_index

    xp, a1 = _k1(x, gat1_W, gat1_as, gat1_ad)
    w1 = _s1(ei, a1)
    msg1 = _s2(xp.reshape(H1 * N, CP), w1, ei)
    hp2, a2 = _k2(msg1, a1, xp, gat1_b.reshape(1, -1), gat2_W,
                  gat2_as, gat2_ad)
    w2, s2p = _s3(ei, a2)
    msg2h = _s4(hp2.reshape(2 * N, 64), w2, ei)
    gpart = _s5(msg2h, hp2, a2, s2p, batch, gat2_b)

    t = _kcnn(target.reshape(B, 1, L), emb,
              jnp.transpose(c1_W, (2, 1, 0)), c1_b.reshape(1, -1),
              jnp.transpose(c2_W, (2, 1, 0)), c2_b.reshape(1, -1),
              jnp.transpose(c3_W, (2, 1, 0)), c3_b.reshape(1, -1)
              ).reshape(B, 128)

    return _khead(gpart, t, fcg_W, fcg_b.reshape(1, -1), fcp_W,
                  fcp_b.reshape(1, -1), fc1_W, fc1_b.reshape(1, -1),
                  fc2_W, fc2_b.reshape(1, -1), fc3_W, fc3_b.reshape(1, -1),
                  outW, outb.reshape(1, -1))


# speedup vs baseline: 15.5724x; 1.4055x over previous
import jax
import jax.numpy as jnp
from jax import lax
from jax.experimental import pallas as pl
from jax.experimental.pallas import tpu as pltpu
from jax.experimental.pallas import tpu_sc as plsc

N = 10000
E = 160000
F_IN = 78
H1 = 10
C1 = 78
CP = 80
OUT_DIM = 128
B = 128
L = 1000
VOCAB = 27
EMB = 128

NC = 2
NS = 16
NW = NC * NS
KC = 128
NCHR = E // KC
NCH = 1280
EP = NCH * KC
NP = 10240
ROWS_PER_TILE = NP // NS

_SC_PARAMS = pltpu.CompilerParams(needs_layout_passes=False,
                                  use_tc_tiling_on_sc=False)


def _mesh():
    return plsc.VectorSubcoreMesh(core_axis_name="c", subcore_axis_name="s",
                                  num_cores=NC, num_subcores=NS)


def _iota16():
    return lax.iota(jnp.int32, 16)


def _splat(i):
    return jnp.full((16,), i, jnp.int32)


def _lrelu_exp(v):
    return jnp.exp(jnp.maximum(v, 0.2 * v))


def _zero16():
    return jnp.zeros((16,), jnp.float32)


def _k1_body(x_ref, w_ref, as_ref, ad_ref, xp_ref, a1_ref):
    x = x_ref[...]
    w = w_ref[...]
    xp = jnp.dot(x, w, preferred_element_type=jnp.float32)
    r = x.shape[0]
    srcs = []
    dsts = []
    for h in range(H1):
        xph = xp[:, h * C1:(h + 1) * C1]
        xp_ref[h, :, 0:C1] = xph
        xp_ref[h, :, C1:CP] = jnp.zeros((r, CP - C1), jnp.float32)
        srcs.append(jnp.sum(xph * as_ref[h, :][None, :], axis=1, keepdims=True))
        dsts.append(jnp.sum(xph * ad_ref[h, :][None, :], axis=1, keepdims=True))
    z6 = jnp.zeros((r, 6), jnp.float32)
    a1_ref[...] = jnp.concatenate(srcs + [z6] + dsts + [z6], axis=1)


def _k1(x, gat1_W, gat1_as, gat1_ad):
    nblk = 10
    r = N // nblk
    return pl.pallas_call(
        _k1_body,
        grid=(nblk,),
        in_specs=[
            pl.BlockSpec((r, F_IN), lambda i: (i, 0)),
            pl.BlockSpec((F_IN, H1 * C1), lambda i: (0, 0)),
            pl.BlockSpec((H1, C1), lambda i: (0, 0)),
            pl.BlockSpec((H1, C1), lambda i: (0, 0)),
        ],
        out_specs=[
            pl.BlockSpec((H1, r, CP), lambda i: (0, i, 0)),
            pl.BlockSpec((r, 32), lambda i: (i, 0)),
        ],
        out_shape=[
            jax.ShapeDtypeStruct((H1, N, CP), jnp.float32),
            jax.ShapeDtypeStruct((N, 32), jnp.float32),
        ],
    )(x, gat1_W, gat1_as, gat1_ad)


def _s1_body(ei3, a1, w1, srcall, dstall, asb0, adb0, asb1, adb1, wt10,
             sg0, sg1):
    cid = lax.axis_index("c")
    sid = lax.axis_index("s")
    wid = sid * NC + cid
    c0 = wid * 40

    pltpu.sync_copy(ei3.at[0, pl.ds(c0, 40)], srcall)
    pltpu.sync_copy(ei3.at[1, pl.ds(c0, 40)], dstall)

    bufs = ((asb0, adb0, sg0), (asb1, adb1, sg1))

    def _issue(j, b):
        asb, adb, sg = bufs[b]

        @pl.when(c0 + j < NCHR)
        def _():
            pltpu.async_copy(a1.at[srcall.at[j]], asb, sg)
            pltpu.async_copy(a1.at[dstall.at[j]], adb, sg)

    def _wait(j, b):
        asb, adb, sg = bufs[b]

        @pl.when(c0 + j < NCHR)
        def _():
            pltpu.make_async_copy(a1.at[srcall.at[j]], asb, sg).wait()
            pltpu.make_async_copy(a1.at[dstall.at[j]], adb, sg).wait()

    _issue(0, 0)
    _issue(1, 1)

    def _outer(t, _):
        for k in range(2):
            j = 2 * t + k
            asb, adb, _sg = bufs[k]
            _wait(j, k)

            jm = j - (j // 10) * 10

            @pl.when(c0 + j < NCHR)
            def _():
                it = _iota16()
                for g in range(KC // 16):
                    rows = it + (16 * g)
                    for h in range(H1):
                        va = plsc.load_gather(asb, [rows, _splat(h)])
                        vb = plsc.load_gather(adb, [rows, _splat(16 + h)])
                        wt10[jm, h, pl.ds(16 * g, 16)] = _lrelu_exp(va + vb)

            @pl.when(j + 2 < 40)
            def _():
                _issue(j + 2, k)

            @pl.when(c0 + j >= NCHR)
            def _():
                def _zz(rr, _c):
                    for g in range(KC // 16):
                        wt10[jm, rr, pl.ds(16 * g, 16)] = _zero16()
                    return 0
                lax.fori_loop(0, H1, _zz, 0)

            @pl.when(jm == 9)
            def _():
                pltpu.sync_copy(wt10, w1.at[pl.ds(c0 + j - 9, 10)])
        return 0

    lax.fori_loop(0, 20, _outer, 0)


def _s1(ei3, a1):
    return pl.kernel(
        _s1_body,
        out_type=jax.ShapeDtypeStruct((NCH, 16, KC), jnp.float32),
        mesh=_mesh(),
        compiler_params=_SC_PARAMS,
        scratch_types=[
            pltpu.VMEM((40, KC), jnp.int32),
            pltpu.VMEM((40, KC), jnp.int32),
            pltpu.VMEM((KC, 32), jnp.float32),
            pltpu.VMEM((KC, 32), jnp.float32),
            pltpu.VMEM((KC, 32), jnp.float32),
            pltpu.VMEM((KC, 32), jnp.float32),
            pltpu.VMEM((10, 16, KC), jnp.float32),
            pltpu.SemaphoreType.DMA,
            pltpu.SemaphoreType.DMA,
        ],
    )(ei3, a1)


def _s2_body(xp_flat, w1, ei3, msg1, srcall, dstall,
             gi0, gi1, gi2, gi3, rb0, rb1, rb2, rb3, wb0, wb1, wb2, wb3,
             zbuf, acc,
             sg0, sg1, sg2, sg3, ss0, ss1, ss2, ss3):
    cid = lax.axis_index("c")
    sid = lax.axis_index("s")
    c0 = sid * 80
    r0 = sid * ROWS_PER_TILE

    pltpu.sync_copy(ei3.at[0, pl.ds(c0, 80)], srcall)
    pltpu.sync_copy(ei3.at[1, pl.ds(c0, 80)], dstall)

    def _zb(i, _):
        for p in range(CP // 16):
            zbuf[i, pl.ds(16 * p, 16)] = _zero16()
        return 0
    lax.fori_loop(0, ROWS_PER_TILE // 4, _zb, 0)
    for q in range(4):
        pltpu.sync_copy(zbuf, acc.at[pl.ds(r0 + q * (ROWS_PER_TILE // 4),
                                           ROWS_PER_TILE // 4)])
    plsc.subcore_barrier()

    gis = (gi0, gi1, gi2, gi3)
    rbs = (rb0, rb1, rb2, rb3)
    wbs = (wb0, wb1, wb2, wb3)
    sgs = (sg0, sg1, sg2, sg3)
    sss = (ss0, ss1, ss2, ss3)

    for hl in range(H1 // NC):
        h = cid * (H1 // NC) + hl
        base = h * N

        def _issue(j, b):
            gi, rb, wb, sg = gis[b], rbs[b], wbs[b], sgs[b]
            for g in range(KC // 16):
                gi[pl.ds(16 * g, 16)] = srcall[j, pl.ds(16 * g, 16)] + base
            pltpu.async_copy(xp_flat.at[gi], rb, sg)
            pltpu.async_copy(w1.at[c0 + j, h], wb, sg)

        def _wait(j, b):
            gi, rb, wb, sg = gis[b], rbs[b], wbs[b], sgs[b]
            pltpu.make_async_copy(xp_flat.at[gi], rb, sg).wait()
            pltpu.make_async_copy(w1.at[c0 + j, h], wb, sg).wait()

        _issue(0, 0)
        _issue(1, 1)

        def _outer(t, _):
            for k in range(4):
                j = 4 * t + k
                rb, wb = rbs[k], wbs[k]
                _wait(j, k)

                nb = (k + 2) - ((k + 2) // 4) * 4

                @pl.when(jnp.logical_and(j + 2 < 80, j >= 2))
                def _():
                    pltpu.make_async_copy(
                        rbs[nb], acc.at[dstall.at[j - 2]], sss[nb]).wait()

                @pl.when(j + 2 < 80)
                def _():
                    _issue(j + 2, nb)

                it16 = _iota16()

                @plsc.parallel_loop(0, KC, unroll=8)
                def _edge(i):
                    ws = plsc.load_gather(wb, [_splat(i)])
                    for p in range(CP // 16 - 1):
                        rb[i, pl.ds(16 * p, 16)] = (
                            rb[i, pl.ds(16 * p, 16)] * ws)
                    pv = rb[i, pl.ds(CP - 16, 16)] * ws
                    rb[i, pl.ds(CP - 16, 16)] = jnp.where(
                        it16 == (C1 - (CP - 16)), ws, pv)

                pltpu.async_copy(rb, acc.at[dstall.at[j]], sss[k], add=True)
            return 0

        lax.fori_loop(0, 20, _outer, 0)
        for k in range(4):
            pltpu.make_async_copy(
                rbs[k], acc.at[dstall.at[76 + k]], sss[k]).wait()
        plsc.subcore_barrier()
        pltpu.sync_copy(acc.at[pl.ds(r0, ROWS_PER_TILE)],
                        msg1.at[h, pl.ds(r0, ROWS_PER_TILE)])
        if hl + 1 < H1 // NC:
            for q in range(4):
                pltpu.sync_copy(
                    zbuf, acc.at[pl.ds(r0 + q * (ROWS_PER_TILE // 4),
                                       ROWS_PER_TILE // 4)])
        plsc.subcore_barrier()


def _s2(xp_flat, w1, ei3):
    return pl.kernel(
        _s2_body,
        out_type=jax.ShapeDtypeStruct((H1, NP, CP), jnp.float32),
        mesh=_mesh(),
        compiler_params=_SC_PARAMS,
        scratch_types=[
            pltpu.VMEM((80, KC), jnp.int32),
            pltpu.VMEM((80, KC), jnp.int32),
            pltpu.VMEM((KC,), jnp.int32),
            pltpu.VMEM((KC,), jnp.int32),
            pltpu.VMEM((KC,), jnp.int32),
            pltpu.VMEM((KC,), jnp.int32),
            pltpu.VMEM((KC, CP), jnp.float32),
            pltpu.VMEM((KC, CP), jnp.float32),
            pltpu.VMEM((KC, CP), jnp.float32),
            pltpu.VMEM((KC, CP), jnp.float32),
            pltpu.VMEM((KC,), jnp.float32),
            pltpu.VMEM((KC,), jnp.float32),
            pltpu.VMEM((KC,), jnp.float32),
            pltpu.VMEM((KC,), jnp.float32),
            pltpu.VMEM((ROWS_PER_TILE // 4, CP), jnp.float32),
            pltpu.VMEM_SHARED((NP, CP), jnp.float32),
            pltpu.SemaphoreType.DMA,
            pltpu.SemaphoreType.DMA,
            pltpu.SemaphoreType.DMA,
            pltpu.SemaphoreType.DMA,
            pltpu.SemaphoreType.DMA,
            pltpu.SemaphoreType.DMA,
            pltpu.SemaphoreType.DMA,
            pltpu.SemaphoreType.DMA,
        ],
    )(xp_flat, w1, ei3)


def _k2_body(msg1_ref, a1_ref, xp_ref, b1_ref, w2_ref, as2_ref,
             ad2_ref, hp2_ref, a2_ref):
    a1s = a1_ref[:, 0:H1]
    a1d = a1_ref[:, 16:16 + H1]
    wself = _lrelu_exp(a1s + a1d)
    parts = []
    for h in range(H1):
        msg = msg1_ref[h, :, 0:C1]
        xph = xp_ref[h, :, 0:C1]
        sh = msg1_ref[h, :, C1:C1 + 1] + wself[:, h:h + 1]
        num = msg + wself[:, h:h + 1] * xph
        hv = num / sh + b1_ref[0, h * C1:(h + 1) * C1][None, :]
        parts.append(jnp.where(hv > 0, hv, jnp.exp(jnp.minimum(hv, 0.0)) - 1.0))
    h1 = jnp.concatenate(parts, axis=1)
    hp2 = jnp.dot(h1, w2_ref[...], preferred_element_type=jnp.float32)
    hp2_ref[...] = hp2
    asrc = jnp.sum(hp2 * as2_ref[0, :][None, :], axis=1, keepdims=True)
    adst = jnp.sum(hp2 * ad2_ref[0, :][None, :], axis=1, keepdims=True)
    wself2 = _lrelu_exp(asrc + adst)
    r = hp2.shape[0]
    a2_ref[...] = jnp.concatenate(
        [asrc, adst, wself2, jnp.zeros((r, 13), jnp.float32)], axis=1)


def _k2(msg1, a1, xp, gat1_b, gat2_W, gat2_as, gat2_ad):
    nblk = 10
    r = N // nblk
    return pl.pallas_call(
        _k2_body,
        grid=(nblk,),
        in_specs=[
            pl.BlockSpec((H1, r, CP), lambda i: (0, i, 0)),
            pl.BlockSpec((r, 32), lambda i: (i, 0)),
            pl.BlockSpec((H1, r, CP), lambda i: (0, i, 0)),
            pl.BlockSpec((1, H1 * C1), lambda i: (0, 0)),
            pl.BlockSpec((H1 * C1, OUT_DIM), lambda i: (0, 0)),
            pl.BlockSpec((1, OUT_DIM), lambda i: (0, 0)),
            pl.BlockSpec((1, OUT_DIM), lambda i: (0, 0)),
        ],
        out_specs=[
            pl.BlockSpec((r, OUT_DIM), lambda i: (i, 0)),
            pl.BlockSpec((r, 16), lambda i: (i, 0)),
        ],
        out_shape=[
            jax.ShapeDtypeStruct((N, OUT_DIM), jnp.float32),
            jax.ShapeDtypeStruct((N, 16), jnp.float32),
        ],
    )(msg1, a1, xp, gat1_b, gat2_W, gat2_as, gat2_ad)


def _s3_body(ei3, a2, w2, s2p, srcall, dstall, g10, g20, g11, g21, wt40,
             sb0, sb1, zbuf, s_acc, sg0, sg1, ss0, ss1):
    cid = lax.axis_index("c")
    sid = lax.axis_index("s")
    wid = sid * NC + cid
    c0 = wid * 40
    r0 = sid * ROWS_PER_TILE

    def _z(i, _):
        zbuf[i, :] = _zero16()
        return 0
    lax.fori_loop(0, ROWS_PER_TILE, _z, 0)
    pltpu.sync_copy(zbuf, s_acc.at[pl.ds(r0, ROWS_PER_TILE)])
    plsc.subcore_barrier()

    pltpu.sync_copy(ei3.at[0, pl.ds(c0, 40)], srcall)
    pltpu.sync_copy(ei3.at[1, pl.ds(c0, 40)], dstall)

    gbufs = ((g10, g20, sg0), (g11, g21, sg1))
    sbufs = (sb0, sb1)
    sss = (ss0, ss1)

    def _issue(j, b):
        g1, g2, sg = gbufs[b]

        @pl.when(c0 + j < NCHR)
        def _():
            pltpu.async_copy(a2.at[srcall.at[j]], g1, sg)
            pltpu.async_copy(a2.at[dstall.at[j]], g2, sg)

    def _wait(j, b):
        g1, g2, sg = gbufs[b]

        @pl.when(c0 + j < NCHR)
        def _():
            pltpu.make_async_copy(a2.at[srcall.at[j]], g1, sg).wait()
            pltpu.make_async_copy(a2.at[dstall.at[j]], g2, sg).wait()

    _issue(0, 0)
    _issue(1, 1)

    def _outer(t, _):
        for k in range(2):
            j = 2 * t + k
            g1, g2, _sg = gbufs[k]
            sb = sbufs[k]
            _wait(j, k)

            @pl.when(c0 + j < NCHR)
            def _():
                @pl.when(j >= 2)
                def _():
                    pltpu.make_async_copy(
                        sb, s_acc.at[dstall.at[j - 2]], sss[k]).wait()

                it = _iota16()

                def _zs(i, _):
                    sb[i, :] = _zero16()
                    return 0
                lax.fori_loop(0, KC, _zs, 0)

                for g in range(KC // 16):
                    rows = it + (16 * g)
                    va = plsc.load_gather(g1, [rows, _splat(0)])
                    vb = plsc.load_gather(g2, [rows, _splat(1)])
                    wv = _lrelu_exp(va + vb)
                    wt40[j, pl.ds(16 * g, 16)] = wv
                    plsc.store_scatter(sb, [rows, _splat(0)], wv)
                pltpu.async_copy(sb, s_acc.at[dstall.at[j]], sss[k], add=True)

            @pl.when(j + 2 < 40)
            def _():
                _issue(j + 2, k)

            @pl.when(c0 + j >= NCHR)
            def _():
                for g in range(KC // 16):
                    wt40[j, pl.ds(16 * g, 16)] = _zero16()
        return 0

    lax.fori_loop(0, 20, _outer, 0)
    pltpu.sync_copy(wt40, w2.at[pl.ds(c0, 40)])
    for k in range(2):
        pltpu.make_async_copy(
            sbufs[k], s_acc.at[dstall.at[k]], sss[k]).wait()
    plsc.subcore_barrier()
    pltpu.sync_copy(s_acc.at[pl.ds(r0, ROWS_PER_TILE)],
                    s2p.at[cid, pl.ds(r0, ROWS_PER_TILE)])


def _s3(ei3, a2):
    return pl.kernel(
        _s3_body,
        out_type=[
            jax.ShapeDtypeStruct((NCH, KC), jnp.float32),
            jax.ShapeDtypeStruct((NC, NP, 16), jnp.float32),
        ],
        mesh=_mesh(),
        compiler_params=_SC_PARAMS,
        scratch_types=[
            pltpu.VMEM((40, KC), jnp.int32),
            pltpu.VMEM((40, KC), jnp.int32),
            pltpu.VMEM((KC, 16), jnp.float32),
            pltpu.VMEM((KC, 16), jnp.float32),
            pltpu.VMEM((KC, 16), jnp.float32),
            pltpu.VMEM((KC, 16), jnp.float32),
            pltpu.VMEM((40, KC), jnp.float32),
            pltpu.VMEM((KC, 16), jnp.float32),
            pltpu.VMEM((KC, 16), jnp.float32),
            pltpu.VMEM((ROWS_PER_TILE, 16), jnp.float32),
            pltpu.VMEM_SHARED((NP, 16), jnp.float32),
            pltpu.SemaphoreType.DMA,
            pltpu.SemaphoreType.DMA,
            pltpu.SemaphoreType.DMA,
            pltpu.SemaphoreType.DMA,
        ],
    )(ei3, a2)


def _s4_body(hp2_pairs, w2, ei3, msg2h, srcall, dstall,
             gi0, gi1, gi2, gi3, rb0, rb1, rb2, rb3, wb0, wb1, wb2, wb3,
             zbuf, acc,
             sg0, sg1, sg2, sg3, ss0, ss1, ss2, ss3):
    cid = lax.axis_index("c")
    sid = lax.axis_index("s")
    c0 = sid * 80
    r0 = sid * ROWS_PER_TILE

    pltpu.sync_copy(ei3.at[0, pl.ds(c0, 80)], srcall)
    pltpu.sync_copy(ei3.at[1, pl.ds(c0, 80)], dstall)

    def _zb(i, _):
        for p in range(4):
            zbuf[i, pl.ds(16 * p, 16)] = _zero16()
        return 0
    lax.fori_loop(0, ROWS_PER_TILE // 4, _zb, 0)
    for q in range(4):
        pltpu.sync_copy(zbuf, acc.at[pl.ds(r0 + q * (ROWS_PER_TILE // 4),
                                           ROWS_PER_TILE // 4)])
    plsc.subcore_barrier()

    gis = (gi0, gi1, gi2, gi3)
    rbs = (rb0, rb1, rb2, rb3)
    wbs = (wb0, wb1, wb2, wb3)
    sgs = (sg0, sg1, sg2, sg3)
    sss = (ss0, ss1, ss2, ss3)

    def _issue(j, b):
        gi, rb, wb, sg = gis[b], rbs[b], wbs[b], sgs[b]
        for g in range(KC // 16):
            gi[pl.ds(16 * g, 16)] = srcall[j, pl.ds(16 * g, 16)] * 2 + cid
        pltpu.async_copy(hp2_pairs.at[gi], rb, sg)
        pltpu.async_copy(w2.at[c0 + j], wb, sg)

    def _wait(j, b):
        gi, rb, wb, sg = gis[b], rbs[b], wbs[b], sgs[b]
        pltpu.make_async_copy(hp2_pairs.at[gi], rb, sg).wait()
        pltpu.make_async_copy(w2.at[c0 + j], wb, sg).wait()

    _issue(0, 0)
    _issue(1, 1)

    def _outer(t, _):
        for k in range(4):
            j = 4 * t + k
            rb, wb = rbs[k], wbs[k]
            _wait(j, k)

            nb = (k + 2) - ((k + 2) // 4) * 4

            @pl.when(jnp.logical_and(j + 2 < 80, j >= 2))
            def _():
                pltpu.make_async_copy(
                    rbs[nb], acc.at[dstall.at[j - 2]], sss[nb]).wait()

            @pl.when(j + 2 < 80)
            def _():
                _issue(j + 2, nb)

            @plsc.parallel_loop(0, KC, unroll=8)
            def _edge(i):
                ws = plsc.load_gather(wb, [_splat(i)])
                for p in range(4):
                    rb[i, pl.ds(16 * p, 16)] = rb[i, pl.ds(16 * p, 16)] * ws

            pltpu.async_copy(rb, acc.at[dstall.at[j]], sss[k], add=True)
        return 0

    lax.fori_loop(0, 20, _outer, 0)
    for k in range(4):
        pltpu.make_async_copy(
            rbs[k], acc.at[dstall.at[76 + k]], sss[k]).wait()
    plsc.subcore_barrier()
    pltpu.sync_copy(acc.at[pl.ds(r0, ROWS_PER_TILE)],
                    msg2h.at[cid, pl.ds(r0, ROWS_PER_TILE)])


def _s4(hp2_pairs, w2, ei3):
    return pl.kernel(
        _s4_body,
        out_type=jax.ShapeDtypeStruct((NC, NP, 64), jnp.float32),
        mesh=_mesh(),
        compiler_params=_SC_PARAMS,
        scratch_types=[
            pltpu.VMEM((80, KC), jnp.int32),
            pltpu.VMEM((80, KC), jnp.int32),
            pltpu.VMEM((KC,), jnp.int32),
            pltpu.VMEM((KC,), jnp.int32),
            pltpu.VMEM((KC,), jnp.int32),
            pltpu.VMEM((KC,), jnp.int32),
            pltpu.VMEM((KC, 64), jnp.float32),
            pltpu.VMEM((KC, 64), jnp.float32),
            pltpu.VMEM((KC, 64), jnp.float32),
            pltpu.VMEM((KC, 64), jnp.float32),
            pltpu.VMEM((KC,), jnp.float32),
            pltpu.VMEM((KC,), jnp.float32),
            pltpu.VMEM((KC,), jnp.float32),
            pltpu.VMEM((KC,), jnp.float32),
            pltpu.VMEM((ROWS_PER_TILE // 4, 64), jnp.float32),
            pltpu.VMEM_SHARED((NP, 64), jnp.float32),
            pltpu.SemaphoreType.DMA,
            pltpu.SemaphoreType.DMA,
            pltpu.SemaphoreType.DMA,
            pltpu.SemaphoreType.DMA,
            pltpu.SemaphoreType.DMA,
            pltpu.SemaphoreType.DMA,
            pltpu.SemaphoreType.DMA,
            pltpu.SemaphoreType.DMA,
        ],
    )(hp2_pairs, w2, ei3)


_RCHUNK = 80
_NRCH = N // _RCHUNK


def _s5_body(msg2h, hp2, a2, s2p, batch, b2, gpart, m0c, m1c, hpc, a2c, s0c,
             s1c, bc, bbuf, gloc):
    cid = lax.axis_index("c")
    sid = lax.axis_index("s")
    wid = sid * NC + cid

    pltpu.sync_copy(b2, bbuf)

    def _zg(i, _):
        for p in range(OUT_DIM // 16):
            gloc[i, pl.ds(16 * p, 16)] = _zero16()
        return 0
    lax.fori_loop(0, B, _zg, 0)

    nloop = (_NRCH + NW - 1) // NW

    def _chunk(j, _):
        cidx = wid + NW * j

        @pl.when(cidx < _NRCH)
        def _():
            r0 = cidx * _RCHUNK
            sl = pl.ds(r0, _RCHUNK)
            pltpu.sync_copy(msg2h.at[0, sl], m0c)
            pltpu.sync_copy(msg2h.at[1, sl], m1c)
            pltpu.sync_copy(hp2.at[sl], hpc)
            pltpu.sync_copy(a2.at[sl], a2c)
            pltpu.sync_copy(s2p.at[0, sl], s0c)
            pltpu.sync_copy(s2p.at[1, sl], s1c)
            pltpu.sync_copy(batch.at[sl], bc)

            it = _iota16()

            def _row(i, _):
                wself = plsc.load_gather(a2c, [_splat(i), _splat(2)])
                sv = (plsc.load_gather(s0c, [_splat(i), _splat(0)])
                      + plsc.load_gather(s1c, [_splat(i), _splat(0)])
                      + wself)
                rcp = 1.0 / sv
                bid = plsc.load_gather(bc, [_splat(i)])
                for p in range(OUT_DIM // 16):
                    mc = m0c if p < 4 else m1c
                    mslice = mc[i, pl.ds(16 * (p - (p // 4) * 4), 16)]
                    hv = (mslice + wself * hpc[i, pl.ds(16 * p, 16)]) * rcp
                    hv = jnp.maximum(hv + bbuf[pl.ds(16 * p, 16)], 0.0)
                    cols = it + (16 * p)
                    old = plsc.load_gather(gloc, [bid, cols])
                    plsc.store_scatter(gloc, [bid, cols],
                                       jnp.maximum(old, hv))
                return 0
            lax.fori_loop(0, _RCHUNK, _row, 0)
        return 0

    lax.fori_loop(0, nloop, _chunk, 0)
    pltpu.sync_copy(gloc, gpart.at[wid])


def _s5(msg2h, hp2, a2, s2p, batch, gat2_b):
    return pl.kernel(
        _s5_body,
        out_type=jax.ShapeDtypeStruct((NW, B, OUT_DIM), jnp.float32),
        mesh=_mesh(),
        compiler_params=_SC_PARAMS,
        scratch_types=[
            pltpu.VMEM((_RCHUNK, 64), jnp.float32),
            pltpu.VMEM((_RCHUNK, 64), jnp.float32),
            pltpu.VMEM((_RCHUNK, OUT_DIM), jnp.float32),
            pltpu.VMEM((_RCHUNK, 16), jnp.float32),
            pltpu.VMEM((_RCHUNK, 16), jnp.float32),
            pltpu.VMEM((_RCHUNK, 16), jnp.float32),
            pltpu.VMEM((_RCHUNK,), jnp.int32),
            pltpu.VMEM((OUT_DIM,), jnp.float32),
            pltpu.VMEM((B, OUT_DIM), jnp.float32),
        ],
    )(msg2h, hp2, a2, s2p, batch, gat2_b)


def _kcnn_body(t_ref, emb_ref, w1_ref, b1_ref, w2_ref, b2_ref, w3_ref,
               b3_ref, out_ref):
    tids = t_ref[0]
    oh = (lax.broadcasted_iota(jnp.int32, (VOCAB, L), 0) == tids
          ).astype(jnp.float32)
    e = lax.dot_general(emb_ref[...], oh, (((0,), (0,)), ((), ())),
                        preferred_element_type=jnp.float32)

    def conv(xin, w_ref, b_ref, lout):
        acc = jnp.zeros((w_ref.shape[2], lout), jnp.float32)
        for k in range(8):
            acc = acc + lax.dot_general(
                w_ref[k], xin[:, k:k + lout], (((0,), (0,)), ((), ())),
                preferred_element_type=jnp.float32)
        return jnp.maximum(acc + b_ref[0, :][:, None], 0.0)

    y1 = conv(e, w1_ref, b1_ref, L - 7)
    y2 = conv(y1, w2_ref, b2_ref, L - 14)
    y3 = conv(y2, w3_ref, b3_ref, L - 21)
    out_ref[0, 0, :] = jnp.max(y3, axis=1)


def _kcnn(target3, emb, c1_Wk, c1_b, c2_Wk, c2_b, c3_Wk, c3_b):
    return pl.pallas_call(
        _kcnn_body,
        grid=(B,),
        in_specs=[
            pl.BlockSpec((1, 1, L), lambda i: (i, 0, 0)),
            pl.BlockSpec((VOCAB, EMB), lambda i: (0, 0)),
            pl.BlockSpec((8, EMB, 64), lambda i: (0, 0, 0)),
            pl.BlockSpec((1, 64), lambda i: (0, 0)),
            pl.BlockSpec((8, 64, 96), lambda i: (0, 0, 0)),
            pl.BlockSpec((1, 96), lambda i: (0, 0)),
            pl.BlockSpec((8, 96, 128), lambda i: (0, 0, 0)),
            pl.BlockSpec((1, 128), lambda i: (0, 0)),
        ],
        out_specs=pl.BlockSpec((1, 1, 128), lambda i: (i, 0, 0)),
        out_shape=jax.ShapeDtypeStruct((B, 1, 128), jnp.float32),
    )(target3, emb, c1_Wk, c1_b, c2_Wk, c2_b, c3_Wk, c3_b)


def _khead_body(gp_ref, t_ref, fcgw, fcgb, fcpw, fcpb, w1, b1, w2, b2, w3,
                b3, ow, ob, out_ref):
    g = jnp.max(gp_ref[...], axis=0)
    g = jnp.maximum(
        jnp.dot(g, fcgw[...], preferred_element_type=jnp.float32)
        + fcgb[0, :][None, :], 0.0)
    t = jnp.maximum(
        jnp.dot(t_ref[...], fcpw[...], preferred_element_type=jnp.float32)
        + fcpb[0, :][None, :], 0.0)
    xc = jnp.concatenate([g, t], axis=1)
    xc = jnp.maximum(
        jnp.dot(xc, w1[...], preferred_element_type=jnp.float32)
        + b1[0, :][None, :], 0.0)
    xc = jnp.maximum(
        jnp.dot(xc, w2[...], preferred_element_type=jnp.float32)
        + b2[0, :][None, :], 0.0)
    xc = jnp.maximum(
        jnp.dot(xc, w3[...], preferred_element_type=jnp.float32)
        + b3[0, :][None, :], 0.0)
    out_ref[...] = (jnp.dot(xc, ow[...], preferred_element_type=jnp.float32)
                    + ob[0, :][None, :])


def _khead(gpart, t, fcg_W, fcg_b, fcp_W, fcp_b, fc1_W, fc1_b, fc2_W, fc2_b,
           fc3_W, fc3_b, outW, outb):
    return pl.pallas_call(
        _khead_body,
        out_shape=jax.ShapeDtypeStruct((B, 1), jnp.float32),
    )(gpart, t, fcg_W, fcg_b, fcp_W, fcp_b, fc1_W, fc1_b, fc2_W, fc2_b,
      fc3_W, fc3_b, outW, outb)


def kernel(x, edge_index, batch, target, gat1_W, gat1_as, gat1_ad, gat1_b,
           gat2_W, gat2_as, gat2_ad, gat2_b, fcg_W, fcg_b, emb, c1_W, c1_b,
           c2_W, c2_b, c3_W, c3_b, fcp_W, fcp_b, fc1_W, fc1_b, fc2_W, fc2_b,
           fc3_W, fc3_b, outW, outb):
    ei3 = jnp.pad(edge_index, ((0, 0), (0, EP - E))).reshape(2, NCH, KC)

    xp, a1 = _k1(x, gat1_W, gat1_as, gat1_ad)
    w1 = _s1(ei3, a1)
    msg1 = _s2(xp.reshape(H1 * N, CP), w1, ei3)
    hp2, a2 = _k2(msg1, a1, xp, gat1_b.reshape(1, -1), gat2_W,
                  gat2_as, gat2_ad)
    w2, s2p = _s3(ei3, a2)
    msg2h = _s4(hp2.reshape(2 * N, 64), w2, ei3)
    gpart = _s5(msg2h, hp2, a2, s2p, batch, gat2_b)

    t = _kcnn(target.reshape(B, 1, L), emb,
              jnp.transpose(c1_W, (2, 1, 0)), c1_b.reshape(1, -1),
              jnp.transpose(c2_W, (2, 1, 0)), c2_b.reshape(1, -1),
              jnp.transpose(c3_W, (2, 1, 0)), c3_b.reshape(1, -1)
              ).reshape(B, 128)

    return _khead(gpart, t, fcg_W, fcg_b.reshape(1, -1), fcp_W,
                  fcp_b.reshape(1, -1), fc1_W, fc1_b.reshape(1, -1),
                  fc2_W, fc2_b.reshape(1, -1), fc3_W, fc3_b.reshape(1, -1),
                  outW, outb.reshape(1, -1))


# speedup vs baseline: 16.7954x; 1.0785x over previous
import jax
import jax.numpy as jnp
from jax import lax
from jax.experimental import pallas as pl
from jax.experimental.pallas import tpu as pltpu
from jax.experimental.pallas import tpu_sc as plsc

N = 10000
E = 160000
F_IN = 78
H1 = 10
C1 = 78
CP = 80
OUT_DIM = 128
B = 128
L = 1000
VOCAB = 27
EMB = 128

NC = 2
NS = 16
NW = NC * NS
KC = 128
NCHR = E // KC
NCH = 1280
EP = NCH * KC
NP = 10240
ROWS_PER_TILE = NP // NS

_SC_PARAMS = pltpu.CompilerParams(needs_layout_passes=False,
                                  use_tc_tiling_on_sc=False)


def _mesh():
    return plsc.VectorSubcoreMesh(core_axis_name="c", subcore_axis_name="s",
                                  num_cores=NC, num_subcores=NS)


def _iota16():
    return lax.iota(jnp.int32, 16)


def _splat(i):
    return jnp.full((16,), i, jnp.int32)


def _lrelu_exp(v):
    return jnp.exp(jnp.maximum(v, 0.2 * v))


def _zero16():
    return jnp.zeros((16,), jnp.float32)


def _k1_body(x_ref, w_ref, ab_ref, xp_ref, a1_ref):
    x = x_ref[...]
    w = w_ref[...]
    xp = jnp.dot(x, w, preferred_element_type=jnp.float32)
    r = x.shape[0]
    for h in range(H1):
        xph = xp[:, h * C1:(h + 1) * C1]
        xp_ref[h, :, 0:C1] = xph
        xp_ref[h, :, C1:CP] = jnp.zeros((r, CP - C1), jnp.float32)
    a1_ref[...] = jnp.dot(xp, ab_ref[...], preferred_element_type=jnp.float32)


def _k1(x, gat1_W, a_blockdiag):
    nblk = 10
    r = N // nblk
    return pl.pallas_call(
        _k1_body,
        grid=(nblk,),
        in_specs=[
            pl.BlockSpec((r, F_IN), lambda i: (i, 0)),
            pl.BlockSpec((F_IN, H1 * C1), lambda i: (0, 0)),
            pl.BlockSpec((H1 * C1, 32), lambda i: (0, 0)),
        ],
        out_specs=[
            pl.BlockSpec((H1, r, CP), lambda i: (0, i, 0)),
            pl.BlockSpec((r, 32), lambda i: (i, 0)),
        ],
        out_shape=[
            jax.ShapeDtypeStruct((H1, N, CP), jnp.float32),
            jax.ShapeDtypeStruct((N, 32), jnp.float32),
        ],
    )(x, gat1_W, a_blockdiag)


def _s1_body(ei3, a1, w1, srcall, dstall, asb0, adb0, asb1, adb1, wt10,
             sg0, sg1):
    cid = lax.axis_index("c")
    sid = lax.axis_index("s")
    wid = sid * NC + cid
    c0 = wid * 40

    pltpu.sync_copy(ei3.at[0, pl.ds(c0, 40)], srcall)
    pltpu.sync_copy(ei3.at[1, pl.ds(c0, 40)], dstall)

    bufs = ((asb0, adb0, sg0), (asb1, adb1, sg1))

    def _issue(j, b):
        asb, adb, sg = bufs[b]

        @pl.when(c0 + j < NCHR)
        def _():
            pltpu.async_copy(a1.at[srcall.at[j]], asb, sg)
            pltpu.async_copy(a1.at[dstall.at[j]], adb, sg)

    def _wait(j, b):
        asb, adb, sg = bufs[b]

        @pl.when(c0 + j < NCHR)
        def _():
            pltpu.make_async_copy(a1.at[srcall.at[j]], asb, sg).wait()
            pltpu.make_async_copy(a1.at[dstall.at[j]], adb, sg).wait()

    _issue(0, 0)
    _issue(1, 1)

    def _outer(t, _):
        for k in range(2):
            j = 2 * t + k
            asb, adb, _sg = bufs[k]
            _wait(j, k)

            jm = j - (j // 10) * 10

            @pl.when(c0 + j < NCHR)
            def _():
                it = _iota16()
                for g in range(KC // 16):
                    rows = it + (16 * g)
                    for h in range(H1):
                        va = plsc.load_gather(asb, [rows, _splat(h)])
                        vb = plsc.load_gather(adb, [rows, _splat(16 + h)])
                        wt10[jm, h, pl.ds(16 * g, 16)] = _lrelu_exp(va + vb)

            @pl.when(j + 2 < 40)
            def _():
                _issue(j + 2, k)

            @pl.when(c0 + j >= NCHR)
            def _():
                def _zz(rr, _c):
                    for g in range(KC // 16):
                        wt10[jm, rr, pl.ds(16 * g, 16)] = _zero16()
                    return 0
                lax.fori_loop(0, H1, _zz, 0)

            @pl.when(jm == 9)
            def _():
                pltpu.sync_copy(wt10, w1.at[pl.ds(c0 + j - 9, 10)])
        return 0

    lax.fori_loop(0, 20, _outer, 0)


def _s1(ei3, a1):
    return pl.kernel(
        _s1_body,
        out_type=jax.ShapeDtypeStruct((NCH, 16, KC), jnp.float32),
        mesh=_mesh(),
        compiler_params=_SC_PARAMS,
        scratch_types=[
            pltpu.VMEM((40, KC), jnp.int32),
            pltpu.VMEM((40, KC), jnp.int32),
            pltpu.VMEM((KC, 32), jnp.float32),
            pltpu.VMEM((KC, 32), jnp.float32),
            pltpu.VMEM((KC, 32), jnp.float32),
            pltpu.VMEM((KC, 32), jnp.float32),
            pltpu.VMEM((10, 16, KC), jnp.float32),
            pltpu.SemaphoreType.DMA,
            pltpu.SemaphoreType.DMA,
        ],
    )(ei3, a1)


def _s2_body(xp_flat, w1, ei3, msg1, srcall, dstall,
             gi0, gi1, gi2, gi3, rb0, rb1, rb2, rb3, wb0, wb1, wb2, wb3,
             zbuf, acc,
             sg0, sg1, sg2, sg3, ss0, ss1, ss2, ss3):
    cid = lax.axis_index("c")
    sid = lax.axis_index("s")
    c0 = sid * 80
    r0 = sid * ROWS_PER_TILE

    pltpu.sync_copy(ei3.at[0, pl.ds(c0, 80)], srcall)
    pltpu.sync_copy(ei3.at[1, pl.ds(c0, 80)], dstall)

    def _zb(i, _):
        for p in range(CP // 16):
            zbuf[i, pl.ds(16 * p, 16)] = _zero16()
        return 0
    lax.fori_loop(0, ROWS_PER_TILE // 4, _zb, 0)
    for q in range(4):
        pltpu.sync_copy(zbuf, acc.at[pl.ds(r0 + q * (ROWS_PER_TILE // 4),
                                           ROWS_PER_TILE // 4)])
    plsc.subcore_barrier()

    gis = (gi0, gi1, gi2, gi3)
    rbs = (rb0, rb1, rb2, rb3)
    wbs = (wb0, wb1, wb2, wb3)
    sgs = (sg0, sg1, sg2, sg3)
    sss = (ss0, ss1, ss2, ss3)

    for hl in range(H1 // NC):
        h = cid * (H1 // NC) + hl
        base = h * N

        def _issue(j, b):
            gi, rb, wb, sg = gis[b], rbs[b], wbs[b], sgs[b]
            for g in range(KC // 16):
                gi[pl.ds(16 * g, 16)] = srcall[j, pl.ds(16 * g, 16)] + base
            pltpu.async_copy(xp_flat.at[gi], rb, sg)
            pltpu.async_copy(w1.at[c0 + j, h], wb, sg)

        def _wait(j, b):
            gi, rb, wb, sg = gis[b], rbs[b], wbs[b], sgs[b]
            pltpu.make_async_copy(xp_flat.at[gi], rb, sg).wait()
            pltpu.make_async_copy(w1.at[c0 + j, h], wb, sg).wait()

        _issue(0, 0)
        _issue(1, 1)

        def _outer(t, _):
            for k in range(4):
                j = 4 * t + k
                rb, wb = rbs[k], wbs[k]
                _wait(j, k)

                nb = (k + 2) - ((k + 2) // 4) * 4

                @pl.when(jnp.logical_and(j + 2 < 80, j >= 2))
                def _():
                    pltpu.make_async_copy(
                        rbs[nb], acc.at[dstall.at[j - 2]], sss[nb]).wait()

                @pl.when(j + 2 < 80)
                def _():
                    _issue(j + 2, nb)

                it16 = _iota16()

                @plsc.parallel_loop(0, KC // 2, unroll=4)
                def _edge(i2):
                    for ii in range(2):
                        i = i2 * 2 + ii
                        ws = plsc.load_gather(wb, [_splat(i)])
                        for p in range(CP // 16 - 1):
                            rb[i, pl.ds(16 * p, 16)] = (
                                rb[i, pl.ds(16 * p, 16)] * ws)
                        pv = rb[i, pl.ds(CP - 16, 16)] * ws
                        rb[i, pl.ds(CP - 16, 16)] = jnp.where(
                            it16 == (C1 - (CP - 16)), ws, pv)

                pltpu.async_copy(rb, acc.at[dstall.at[j]], sss[k], add=True)
            return 0

        lax.fori_loop(0, 20, _outer, 0)
        for k in range(4):
            pltpu.make_async_copy(
                rbs[k], acc.at[dstall.at[76 + k]], sss[k]).wait()
        plsc.subcore_barrier()
        pltpu.sync_copy(acc.at[pl.ds(r0, ROWS_PER_TILE)],
                        msg1.at[h, pl.ds(r0, ROWS_PER_TILE)])
        if hl + 1 < H1 // NC:
            for q in range(4):
                pltpu.sync_copy(
                    zbuf, acc.at[pl.ds(r0 + q * (ROWS_PER_TILE // 4),
                                       ROWS_PER_TILE // 4)])
        plsc.subcore_barrier()


def _s2(xp_flat, w1, ei3):
    return pl.kernel(
        _s2_body,
        out_type=jax.ShapeDtypeStruct((H1, NP, CP), jnp.float32),
        mesh=_mesh(),
        compiler_params=_SC_PARAMS,
        scratch_types=[
            pltpu.VMEM((80, KC), jnp.int32),
            pltpu.VMEM((80, KC), jnp.int32),
            pltpu.VMEM((KC,), jnp.int32),
            pltpu.VMEM((KC,), jnp.int32),
            pltpu.VMEM((KC,), jnp.int32),
            pltpu.VMEM((KC,), jnp.int32),
            pltpu.VMEM((KC, CP), jnp.float32),
            pltpu.VMEM((KC, CP), jnp.float32),
            pltpu.VMEM((KC, CP), jnp.float32),
            pltpu.VMEM((KC, CP), jnp.float32),
            pltpu.VMEM((KC,), jnp.float32),
            pltpu.VMEM((KC,), jnp.float32),
            pltpu.VMEM((KC,), jnp.float32),
            pltpu.VMEM((KC,), jnp.float32),
            pltpu.VMEM((ROWS_PER_TILE // 4, CP), jnp.float32),
            pltpu.VMEM_SHARED((NP, CP), jnp.float32),
            pltpu.SemaphoreType.DMA,
            pltpu.SemaphoreType.DMA,
            pltpu.SemaphoreType.DMA,
            pltpu.SemaphoreType.DMA,
            pltpu.SemaphoreType.DMA,
            pltpu.SemaphoreType.DMA,
            pltpu.SemaphoreType.DMA,
            pltpu.SemaphoreType.DMA,
        ],
    )(xp_flat, w1, ei3)


def _k2_body(msg1_ref, a1_ref, xp_ref, b1_ref, w2_ref, as2_ref,
             ad2_ref, hp2_ref, a2_ref):
    a1s = a1_ref[:, 0:H1]
    a1d = a1_ref[:, 16:16 + H1]
    wself = _lrelu_exp(a1s + a1d)
    parts = []
    for h in range(H1):
        msg = msg1_ref[h, :, 0:C1]
        xph = xp_ref[h, :, 0:C1]
        sh = msg1_ref[h, :, C1:C1 + 1] + wself[:, h:h + 1]
        num = msg + wself[:, h:h + 1] * xph
        hv = num / sh + b1_ref[0, h * C1:(h + 1) * C1][None, :]
        parts.append(jnp.where(hv > 0, hv, jnp.exp(jnp.minimum(hv, 0.0)) - 1.0))
    h1 = jnp.concatenate(parts, axis=1)
    hp2 = jnp.dot(h1, w2_ref[...], preferred_element_type=jnp.float32)
    hp2_ref[...] = hp2
    asrc = jnp.sum(hp2 * as2_ref[0, :][None, :], axis=1, keepdims=True)
    adst = jnp.sum(hp2 * ad2_ref[0, :][None, :], axis=1, keepdims=True)
    wself2 = _lrelu_exp(asrc + adst)
    r = hp2.shape[0]
    a2_ref[...] = jnp.concatenate(
        [asrc, adst, wself2, jnp.zeros((r, 13), jnp.float32)], axis=1)


def _k2(msg1, a1, xp, gat1_b, gat2_W, gat2_as, gat2_ad):
    nblk = 10
    r = N // nblk
    return pl.pallas_call(
        _k2_body,
        grid=(nblk,),
        in_specs=[
            pl.BlockSpec((H1, r, CP), lambda i: (0, i, 0)),
            pl.BlockSpec((r, 32), lambda i: (i, 0)),
            pl.BlockSpec((H1, r, CP), lambda i: (0, i, 0)),
            pl.BlockSpec((1, H1 * C1), lambda i: (0, 0)),
            pl.BlockSpec((H1 * C1, OUT_DIM), lambda i: (0, 0)),
            pl.BlockSpec((1, OUT_DIM), lambda i: (0, 0)),
            pl.BlockSpec((1, OUT_DIM), lambda i: (0, 0)),
        ],
        out_specs=[
            pl.BlockSpec((r, OUT_DIM), lambda i: (i, 0)),
            pl.BlockSpec((r, 16), lambda i: (i, 0)),
        ],
        out_shape=[
            jax.ShapeDtypeStruct((N, OUT_DIM), jnp.float32),
            jax.ShapeDtypeStruct((N, 16), jnp.float32),
        ],
    )(msg1, a1, xp, gat1_b, gat2_W, gat2_as, gat2_ad)


def _s3_body(ei3, a2, w2, s2p, srcall, dstall, g10, g20, g11, g21, wt40,
             sb0, sb1, zbuf, s_acc, sg0, sg1, ss0, ss1):
    cid = lax.axis_index("c")
    sid = lax.axis_index("s")
    wid = sid * NC + cid
    c0 = wid * 40
    r0 = sid * ROWS_PER_TILE

    def _z(i, _):
        zbuf[i, :] = _zero16()
        return 0
    lax.fori_loop(0, ROWS_PER_TILE, _z, 0)
    pltpu.sync_copy(zbuf, s_acc.at[pl.ds(r0, ROWS_PER_TILE)])
    plsc.subcore_barrier()

    pltpu.sync_copy(ei3.at[0, pl.ds(c0, 40)], srcall)
    pltpu.sync_copy(ei3.at[1, pl.ds(c0, 40)], dstall)

    gbufs = ((g10, g20, sg0), (g11, g21, sg1))
    sbufs = (sb0, sb1)
    sss = (ss0, ss1)

    def _issue(j, b):
        g1, g2, sg = gbufs[b]

        @pl.when(c0 + j < NCHR)
        def _():
            pltpu.async_copy(a2.at[srcall.at[j]], g1, sg)
            pltpu.async_copy(a2.at[dstall.at[j]], g2, sg)

    def _wait(j, b):
        g1, g2, sg = gbufs[b]

        @pl.when(c0 + j < NCHR)
        def _():
            pltpu.make_async_copy(a2.at[srcall.at[j]], g1, sg).wait()
            pltpu.make_async_copy(a2.at[dstall.at[j]], g2, sg).wait()

    _issue(0, 0)
    _issue(1, 1)

    def _outer(t, _):
        for k in range(2):
            j = 2 * t + k
            g1, g2, _sg = gbufs[k]
            sb = sbufs[k]
            _wait(j, k)

            @pl.when(c0 + j < NCHR)
            def _():
                @pl.when(j >= 2)
                def _():
                    pltpu.make_async_copy(
                        sb, s_acc.at[dstall.at[j - 2]], sss[k]).wait()

                it = _iota16()

                def _zs(i, _):
                    sb[i, :] = _zero16()
                    return 0
                lax.fori_loop(0, KC, _zs, 0)

                for g in range(KC // 16):
                    rows = it + (16 * g)
                    va = plsc.load_gather(g1, [rows, _splat(0)])
                    vb = plsc.load_gather(g2, [rows, _splat(1)])
                    wv = _lrelu_exp(va + vb)
                    wt40[j, pl.ds(16 * g, 16)] = wv
                    plsc.store_scatter(sb, [rows, _splat(0)], wv)
                pltpu.async_copy(sb, s_acc.at[dstall.at[j]], sss[k], add=True)

            @pl.when(j + 2 < 40)
            def _():
                _issue(j + 2, k)

            @pl.when(c0 + j >= NCHR)
            def _():
                for g in range(KC // 16):
                    wt40[j, pl.ds(16 * g, 16)] = _zero16()
        return 0

    lax.fori_loop(0, 20, _outer, 0)
    pltpu.sync_copy(wt40, w2.at[pl.ds(c0, 40)])
    for k in range(2):
        pltpu.make_async_copy(
            sbufs[k], s_acc.at[dstall.at[k]], sss[k]).wait()
    plsc.subcore_barrier()
    pltpu.sync_copy(s_acc.at[pl.ds(r0, ROWS_PER_TILE)],
                    s2p.at[cid, pl.ds(r0, ROWS_PER_TILE)])


def _s3(ei3, a2):
    return pl.kernel(
        _s3_body,
        out_type=[
            jax.ShapeDtypeStruct((NCH, KC), jnp.float32),
            jax.ShapeDtypeStruct((NC, NP, 16), jnp.float32),
        ],
        mesh=_mesh(),
        compiler_params=_SC_PARAMS,
        scratch_types=[
            pltpu.VMEM((40, KC), jnp.int32),
            pltpu.VMEM((40, KC), jnp.int32),
            pltpu.VMEM((KC, 16), jnp.float32),
            pltpu.VMEM((KC, 16), jnp.float32),
            pltpu.VMEM((KC, 16), jnp.float32),
            pltpu.VMEM((KC, 16), jnp.float32),
            pltpu.VMEM((40, KC), jnp.float32),
            pltpu.VMEM((KC, 16), jnp.float32),
            pltpu.VMEM((KC, 16), jnp.float32),
            pltpu.VMEM((ROWS_PER_TILE, 16), jnp.float32),
            pltpu.VMEM_SHARED((NP, 16), jnp.float32),
            pltpu.SemaphoreType.DMA,
            pltpu.SemaphoreType.DMA,
            pltpu.SemaphoreType.DMA,
            pltpu.SemaphoreType.DMA,
        ],
    )(ei3, a2)


def _s4_body(hp2_pairs, w2, ei3, msg2h, srcall, dstall,
             gi0, gi1, gi2, gi3, rb0, rb1, rb2, rb3, wb0, wb1, wb2, wb3,
             zbuf, acc,
             sg0, sg1, sg2, sg3, ss0, ss1, ss2, ss3):
    cid = lax.axis_index("c")
    sid = lax.axis_index("s")
    c0 = sid * 80
    r0 = sid * ROWS_PER_TILE

    pltpu.sync_copy(ei3.at[0, pl.ds(c0, 80)], srcall)
    pltpu.sync_copy(ei3.at[1, pl.ds(c0, 80)], dstall)

    def _zb(i, _):
        for p in range(4):
            zbuf[i, pl.ds(16 * p, 16)] = _zero16()
        return 0
    lax.fori_loop(0, ROWS_PER_TILE // 4, _zb, 0)
    for q in range(4):
        pltpu.sync_copy(zbuf, acc.at[pl.ds(r0 + q * (ROWS_PER_TILE // 4),
                                           ROWS_PER_TILE // 4)])
    plsc.subcore_barrier()

    gis = (gi0, gi1, gi2, gi3)
    rbs = (rb0, rb1, rb2, rb3)
    wbs = (wb0, wb1, wb2, wb3)
    sgs = (sg0, sg1, sg2, sg3)
    sss = (ss0, ss1, ss2, ss3)

    def _issue(j, b):
        gi, rb, wb, sg = gis[b], rbs[b], wbs[b], sgs[b]
        for g in range(KC // 16):
            gi[pl.ds(16 * g, 16)] = srcall[j, pl.ds(16 * g, 16)] * 2 + cid
        pltpu.async_copy(hp2_pairs.at[gi], rb, sg)
        pltpu.async_copy(w2.at[c0 + j], wb, sg)

    def _wait(j, b):
        gi, rb, wb, sg = gis[b], rbs[b], wbs[b], sgs[b]
        pltpu.make_async_copy(hp2_pairs.at[gi], rb, sg).wait()
        pltpu.make_async_copy(w2.at[c0 + j], wb, sg).wait()

    _issue(0, 0)
    _issue(1, 1)

    def _outer(t, _):
        for k in range(4):
            j = 4 * t + k
            rb, wb = rbs[k], wbs[k]
            _wait(j, k)

            nb = (k + 2) - ((k + 2) // 4) * 4

            @pl.when(jnp.logical_and(j + 2 < 80, j >= 2))
            def _():
                pltpu.make_async_copy(
                    rbs[nb], acc.at[dstall.at[j - 2]], sss[nb]).wait()

            @pl.when(j + 2 < 80)
            def _():
                _issue(j + 2, nb)

            @plsc.parallel_loop(0, KC // 2, unroll=4)
            def _edge(i2):
                for ii in range(2):
                    i = i2 * 2 + ii
                    ws = plsc.load_gather(wb, [_splat(i)])
                    for p in range(4):
                        rb[i, pl.ds(16 * p, 16)] = (
                            rb[i, pl.ds(16 * p, 16)] * ws)

            pltpu.async_copy(rb, acc.at[dstall.at[j]], sss[k], add=True)
        return 0

    lax.fori_loop(0, 20, _outer, 0)
    for k in range(4):
        pltpu.make_async_copy(
            rbs[k], acc.at[dstall.at[76 + k]], sss[k]).wait()
    plsc.subcore_barrier()
    pltpu.sync_copy(acc.at[pl.ds(r0, ROWS_PER_TILE)],
                    msg2h.at[cid, pl.ds(r0, ROWS_PER_TILE)])


def _s4(hp2_pairs, w2, ei3):
    return pl.kernel(
        _s4_body,
        out_type=jax.ShapeDtypeStruct((NC, NP, 64), jnp.float32),
        mesh=_mesh(),
        compiler_params=_SC_PARAMS,
        scratch_types=[
            pltpu.VMEM((80, KC), jnp.int32),
            pltpu.VMEM((80, KC), jnp.int32),
            pltpu.VMEM((KC,), jnp.int32),
            pltpu.VMEM((KC,), jnp.int32),
            pltpu.VMEM((KC,), jnp.int32),
            pltpu.VMEM((KC,), jnp.int32),
            pltpu.VMEM((KC, 64), jnp.float32),
            pltpu.VMEM((KC, 64), jnp.float32),
            pltpu.VMEM((KC, 64), jnp.float32),
            pltpu.VMEM((KC, 64), jnp.float32),
            pltpu.VMEM((KC,), jnp.float32),
            pltpu.VMEM((KC,), jnp.float32),
            pltpu.VMEM((KC,), jnp.float32),
            pltpu.VMEM((KC,), jnp.float32),
            pltpu.VMEM((ROWS_PER_TILE // 4, 64), jnp.float32),
            pltpu.VMEM_SHARED((NP, 64), jnp.float32),
            pltpu.SemaphoreType.DMA,
            pltpu.SemaphoreType.DMA,
            pltpu.SemaphoreType.DMA,
            pltpu.SemaphoreType.DMA,
            pltpu.SemaphoreType.DMA,
            pltpu.SemaphoreType.DMA,
            pltpu.SemaphoreType.DMA,
            pltpu.SemaphoreType.DMA,
        ],
    )(hp2_pairs, w2, ei3)


_RCHUNK = 80
_NRCH = N // _RCHUNK


def _s5_body(msg2h, hp2, a2, s2p, batch, b2, gpart, m0c, m1c, hpc, a2c, s0c,
             s1c, bc, bbuf, gloc):
    cid = lax.axis_index("c")
    sid = lax.axis_index("s")
    wid = sid * NC + cid

    pltpu.sync_copy(b2, bbuf)

    def _zg(i, _):
        for p in range(OUT_DIM // 16):
            gloc[i, pl.ds(16 * p, 16)] = _zero16()
        return 0
    lax.fori_loop(0, B, _zg, 0)

    nloop = (_NRCH + NW - 1) // NW

    def _chunk(j, _):
        cidx = wid + NW * j

        @pl.when(cidx < _NRCH)
        def _():
            r0 = cidx * _RCHUNK
            sl = pl.ds(r0, _RCHUNK)
            pltpu.sync_copy(msg2h.at[0, sl], m0c)
            pltpu.sync_copy(msg2h.at[1, sl], m1c)
            pltpu.sync_copy(hp2.at[sl], hpc)
            pltpu.sync_copy(a2.at[sl], a2c)
            pltpu.sync_copy(s2p.at[0, sl], s0c)
            pltpu.sync_copy(s2p.at[1, sl], s1c)
            pltpu.sync_copy(batch.at[sl], bc)

            it = _iota16()

            def _row(i, _):
                wself = plsc.load_gather(a2c, [_splat(i), _splat(2)])
                sv = (plsc.load_gather(s0c, [_splat(i), _splat(0)])
                      + plsc.load_gather(s1c, [_splat(i), _splat(0)])
                      + wself)
                rcp = 1.0 / sv
                bid = plsc.load_gather(bc, [_splat(i)])
                for p in range(OUT_DIM // 16):
                    mc = m0c if p < 4 else m1c
                    mslice = mc[i, pl.ds(16 * (p - (p // 4) * 4), 16)]
                    hv = (mslice + wself * hpc[i, pl.ds(16 * p, 16)]) * rcp
                    hv = jnp.maximum(hv + bbuf[pl.ds(16 * p, 16)], 0.0)
                    cols = it + (16 * p)
                    old = plsc.load_gather(gloc, [bid, cols])
                    plsc.store_scatter(gloc, [bid, cols],
                                       jnp.maximum(old, hv))
                return 0
            lax.fori_loop(0, _RCHUNK, _row, 0)
        return 0

    lax.fori_loop(0, nloop, _chunk, 0)
    pltpu.sync_copy(gloc, gpart.at[wid])


def _s5(msg2h, hp2, a2, s2p, batch, gat2_b):
    return pl.kernel(
        _s5_body,
        out_type=jax.ShapeDtypeStruct((NW, B, OUT_DIM), jnp.float32),
        mesh=_mesh(),
        compiler_params=_SC_PARAMS,
        scratch_types=[
            pltpu.VMEM((_RCHUNK, 64), jnp.float32),
            pltpu.VMEM((_RCHUNK, 64), jnp.float32),
            pltpu.VMEM((_RCHUNK, OUT_DIM), jnp.float32),
            pltpu.VMEM((_RCHUNK, 16), jnp.float32),
            pltpu.VMEM((_RCHUNK, 16), jnp.float32),
            pltpu.VMEM((_RCHUNK, 16), jnp.float32),
            pltpu.VMEM((_RCHUNK,), jnp.int32),
            pltpu.VMEM((OUT_DIM,), jnp.float32),
            pltpu.VMEM((B, OUT_DIM), jnp.float32),
        ],
    )(msg2h, hp2, a2, s2p, batch, gat2_b)


def _kcnn_body(t_ref, emb_ref, w1_ref, b1_ref, w2_ref, b2_ref, w3_ref,
               b3_ref, out_ref):
    tids = t_ref[0]
    oh = (lax.broadcasted_iota(jnp.int32, (VOCAB, L), 0) == tids
          ).astype(jnp.float32)
    e = lax.dot_general(emb_ref[...], oh, (((0,), (0,)), ((), ())),
                        preferred_element_type=jnp.float32)

    def conv(xin, w_ref, b_ref, lout):
        acc = jnp.zeros((w_ref.shape[2], lout), jnp.float32)
        for k in range(8):
            acc = acc + lax.dot_general(
                w_ref[k], xin[:, k:k + lout], (((0,), (0,)), ((), ())),
                preferred_element_type=jnp.float32)
        return jnp.maximum(acc + b_ref[0, :][:, None], 0.0)

    y1 = conv(e, w1_ref, b1_ref, L - 7)
    y2 = conv(y1, w2_ref, b2_ref, L - 14)
    y3 = conv(y2, w3_ref, b3_ref, L - 21)
    out_ref[0, 0, :] = jnp.max(y3, axis=1)


def _kcnn(target3, emb, c1_Wk, c1_b, c2_Wk, c2_b, c3_Wk, c3_b):
    return pl.pallas_call(
        _kcnn_body,
        grid=(B,),
        in_specs=[
            pl.BlockSpec((1, 1, L), lambda i: (i, 0, 0)),
            pl.BlockSpec((VOCAB, EMB), lambda i: (0, 0)),
            pl.BlockSpec((8, EMB, 64), lambda i: (0, 0, 0)),
            pl.BlockSpec((1, 64), lambda i: (0, 0)),
            pl.BlockSpec((8, 64, 96), lambda i: (0, 0, 0)),
            pl.BlockSpec((1, 96), lambda i: (0, 0)),
            pl.BlockSpec((8, 96, 128), lambda i: (0, 0, 0)),
            pl.BlockSpec((1, 128), lambda i: (0, 0)),
        ],
        out_specs=pl.BlockSpec((1, 1, 128), lambda i: (i, 0, 0)),
        out_shape=jax.ShapeDtypeStruct((B, 1, 128), jnp.float32),
    )(target3, emb, c1_Wk, c1_b, c2_Wk, c2_b, c3_Wk, c3_b)


def _khead_body(gp_ref, t_ref, fcgw, fcgb, fcpw, fcpb, w1, b1, w2, b2, w3,
                b3, ow, ob, out_ref):
    g = jnp.max(gp_ref[...], axis=0)
    g = jnp.maximum(
        jnp.dot(g, fcgw[...], preferred_element_type=jnp.float32)
        + fcgb[0, :][None, :], 0.0)
    t = jnp.maximum(
        jnp.dot(t_ref[...], fcpw[...], preferred_element_type=jnp.float32)
        + fcpb[0, :][None, :], 0.0)
    xc = jnp.concatenate([g, t], axis=1)
    xc = jnp.maximum(
        jnp.dot(xc, w1[...], preferred_element_type=jnp.float32)
        + b1[0, :][None, :], 0.0)
    xc = jnp.maximum(
        jnp.dot(xc, w2[...], preferred_element_type=jnp.float32)
        + b2[0, :][None, :], 0.0)
    xc = jnp.maximum(
        jnp.dot(xc, w3[...], preferred_element_type=jnp.float32)
        + b3[0, :][None, :], 0.0)
    out_ref[...] = (jnp.dot(xc, ow[...], preferred_element_type=jnp.float32)
                    + ob[0, :][None, :])


def _khead(gpart, t, fcg_W, fcg_b, fcp_W, fcp_b, fc1_W, fc1_b, fc2_W, fc2_b,
           fc3_W, fc3_b, outW, outb):
    return pl.pallas_call(
        _khead_body,
        out_shape=jax.ShapeDtypeStruct((B, 1), jnp.float32),
    )(gpart, t, fcg_W, fcg_b, fcp_W, fcp_b, fc1_W, fc1_b, fc2_W, fc2_b,
      fc3_W, fc3_b, outW, outb)


def kernel(x, edge_index, batch, target, gat1_W, gat1_as, gat1_ad, gat1_b,
           gat2_W, gat2_as, gat2_ad, gat2_b, fcg_W, fcg_b, emb, c1_W, c1_b,
           c2_W, c2_b, c3_W, c3_b, fcp_W, fcp_b, fc1_W, fc1_b, fc2_W, fc2_b,
           fc3_W, fc3_b, outW, outb):
    ei3 = jnp.pad(edge_index, ((0, 0), (0, EP - E))).reshape(2, NCH, KC)

    ab = jnp.zeros((H1 * C1, 32), jnp.float32)
    for h in range(H1):
        ab = ab.at[h * C1:(h + 1) * C1, h].set(gat1_as[h])
        ab = ab.at[h * C1:(h + 1) * C1, 16 + h].set(gat1_ad[h])
    xp, a1 = _k1(x, gat1_W, ab)
    w1 = _s1(ei3, a1)
    msg1 = _s2(xp.reshape(H1 * N, CP), w1, ei3)
    hp2, a2 = _k2(msg1, a1, xp, gat1_b.reshape(1, -1), gat2_W,
                  gat2_as, gat2_ad)
    w2, s2p = _s3(ei3, a2)
    msg2h = _s4(hp2.reshape(2 * N, 64), w2, ei3)
    gpart = _s5(msg2h, hp2, a2, s2p, batch, gat2_b)

    t = _kcnn(target.reshape(B, 1, L), emb,
              jnp.transpose(c1_W, (2, 1, 0)), c1_b.reshape(1, -1),
              jnp.transpose(c2_W, (2, 1, 0)), c2_b.reshape(1, -1),
              jnp.transpose(c3_W, (2, 1, 0)), c3_b.reshape(1, -1)
              ).reshape(B, 128)

    return _khead(gpart, t, fcg_W, fcg_b.reshape(1, -1), fcp_W,
                  fcp_b.reshape(1, -1), fc1_W, fc1_b.reshape(1, -1),
                  fc2_W, fc2_b.reshape(1, -1), fc3_W, fc3_b.reshape(1, -1),
                  outW, outb.reshape(1, -1))
